# Initial kernel scaffold; baseline (speedup 1.0000x reference)
import jax
import jax.numpy as jnp
from jax.experimental import pallas as pl


def kernel(x, edge_index, W1, att_src1, att_dst1, b1, W2, att_src2, att_dst2, b2):
    raise NotImplementedError("write your pallas kernel here")


# speedup vs baseline: 11.5458x; 11.5458x over previous
import jax
import jax.numpy as jnp
from jax import lax
from jax.experimental import pallas as pl
from jax.experimental.pallas import tpu as pltpu
from jax.experimental.pallas import tpu_sc as plsc

N = 10000
E_RAW = 320000
E1 = E_RAW + N
IN_CH = 128
HID = 128
HEADS = 4
OUT_CH = 64

NC = 2
NS = 16
L = 16

EPAD = 331776
TPS1 = EPAD // NS
TPS2 = EPAD // (NC * NS)

CH = 576
NCH1 = TPS1 // CH
NCH0 = TPS2 // CH
GB = 64
NGC = CH // GB
GBD = 128
NGD = TPS2 // GBD

ACC_R = 10112
SPT = ACC_R // NS
C1 = 144
C2 = 80

NB = 25
BR = N // NB

_EPS = 1e-30


def _stage_a_body(x_ref, w1_ref, asrc_ref, adst_ref, h1h_ref, as_ref, ad_ref):
    xb = x_ref[...]
    h = lax.dot_general(xb, w1_ref[...], (((1,), (0,)), ((), ())),
                        preferred_element_type=jnp.float32)
    for hh in range(HEADS):
        seg = h[:, hh * HID:(hh + 1) * HID]
        h1h_ref[pl.ds(hh * BR, BR), :] = seg
        as_ref[0, pl.ds(hh, 1), :] = lax.dot_general(
            asrc_ref[pl.ds(hh, 1), :], seg, (((1,), (1,)), ((), ())),
            preferred_element_type=jnp.float32)
        ad_ref[0, pl.ds(hh, 1), :] = lax.dot_general(
            adst_ref[pl.ds(hh, 1), :], seg, (((1,), (1,)), ((), ())),
            preferred_element_type=jnp.float32)


def _stage_a(x, W1, att_src1, att_dst1):
    return pl.pallas_call(
        _stage_a_body,
        grid=(NB,),
        in_specs=[
            pl.BlockSpec((BR, IN_CH), lambda i: (i, 0)),
            pl.BlockSpec((IN_CH, HEADS * HID), lambda i: (0, 0)),
            pl.BlockSpec((HEADS, HID), lambda i: (0, 0)),
            pl.BlockSpec((HEADS, HID), lambda i: (0, 0)),
        ],
        out_specs=[
            pl.BlockSpec((HEADS * BR, HID), lambda i: (i, 0)),
            pl.BlockSpec((1, HEADS, BR), lambda i: (i, 0, 0)),
            pl.BlockSpec((1, HEADS, BR), lambda i: (i, 0, 0)),
        ],
        out_shape=[
            jax.ShapeDtypeStruct((NB * HEADS * BR, HID), jnp.float32),
            jax.ShapeDtypeStruct((NB, HEADS, BR), jnp.float32),
            jax.ShapeDtypeStruct((NB, HEADS, BR), jnp.float32),
        ],
    )(x, W1, att_src1, att_dst1)


def _stage_b0_body(ast, adt, s_h, d_h, wout, asv, adv, sbuf, dbuf, w4):
    c = lax.axis_index("c")
    sid = lax.axis_index("s")
    wid = c * NS + sid
    pltpu.sync_copy(ast.at[pl.ds(0, HEADS * N)], asv.at[pl.ds(0, HEADS * N)])
    pltpu.sync_copy(adt.at[pl.ds(0, HEADS * N)], adv.at[pl.ds(0, HEADS * N)])

    def chunk_body(ch, _):
        base = wid * TPS2 + ch * CH
        pltpu.sync_copy(s_h.at[pl.ds(base, CH)], sbuf)
        pltpu.sync_copy(d_h.at[pl.ds(base, CH)], dbuf)

        def q_body(q, _):
            s16 = sbuf[pl.ds(q * L, L)]
            d16 = dbuf[pl.ds(q * L, L)]
            for hh in range(HEADS):
                off = jnp.full((L,), hh * N, jnp.int32)
                t = (plsc.load_gather(asv, [s16 + off])
                     + plsc.load_gather(adv, [d16 + off]))
                t = jnp.maximum(t, 0.2 * t)
                w4[hh, pl.ds(q * L, L)] = jnp.exp(t)
            return 0

        lax.fori_loop(0, CH // L, q_body, 0)
        for hh in range(HEADS):
            pltpu.sync_copy(w4.at[hh], wout.at[pl.ds(hh * EPAD + base, CH)])
        return 0

    lax.fori_loop(0, NCH0, chunk_body, 0)


def _stage_b0(ast, adt, s_h, d_h):
    mesh = plsc.VectorSubcoreMesh(core_axis_name="c", subcore_axis_name="s",
                                  num_cores=NC, num_subcores=NS)
    f = pl.kernel(
        _stage_b0_body,
        out_type=jax.ShapeDtypeStruct((HEADS * EPAD,), jnp.float32),
        mesh=mesh,
        compiler_params=pltpu.CompilerParams(needs_layout_passes=False,
                                             use_tc_tiling_on_sc=False),
        scratch_types=[
            pltpu.VMEM((HEADS * N + L,), jnp.float32),
            pltpu.VMEM((HEADS * N + L,), jnp.float32),
            pltpu.VMEM((CH,), jnp.int32),
            pltpu.VMEM((CH,), jnp.int32),
            pltpu.VMEM((HEADS, CH), jnp.float32),
        ],
    )
    return f(ast, adt, s_h, d_h)


def _zero_acc(acc, zbuf, sid, cols):
    zv = jnp.zeros((L,), jnp.float32)
    for r in range(8):
        for k in range(cols // L):
            zbuf[r, pl.ds(k * L, L)] = zv
    base = sid * SPT
    for r in range(SPT // 8):
        pltpu.sync_copy(zbuf, acc.at[pl.ds(base + r * 8, 8)])


def _stage_b_body(h1h, s_h, d_h, w_h, out1,
                  acc, sbuf, dbuf, wbuf, gidx, rows, scat, zbuf, sem):
    c = lax.axis_index("c")
    sid = lax.axis_index("s")
    iot = lax.iota(jnp.int32, L)

    for hp in range(2):
        hglob = 2 * c + hp
        _zero_acc(acc, zbuf, sid, C1)
        plsc.subcore_barrier()

        def chunk_body(ch, _):
            base = sid * TPS1 + ch * CH
            pltpu.sync_copy(s_h.at[pl.ds(base, CH)], sbuf)
            pltpu.sync_copy(d_h.at[pl.ds(base, CH)], dbuf)
            pltpu.sync_copy(w_h.at[pl.ds(hglob * EPAD + base, CH)], wbuf)
            hoff = hglob * N

            def group_body(g, _):
                for q in range(GB // L):
                    s16 = sbuf[pl.ds(g * GB + q * L, L)]
                    gidx[pl.ds(q * L, L)] = s16 + hoff
                pltpu.async_copy(h1h.at[gidx], rows, sem).wait()

                def scale_body(j, _):
                    wsp = plsc.load_gather(
                        wbuf, [jnp.full((L,), 0, jnp.int32) + (g * GB + j)])
                    for k in range(HID // L):
                        scat[j, pl.ds(k * L, L)] = rows[j, pl.ds(k * L, L)] * wsp
                    scat[j, pl.ds(HID, L)] = jnp.where(iot == 0, wsp, 0.0)
                    return 0

                lax.fori_loop(0, GB, scale_body, 0)
                pltpu.sync_copy(scat, acc.at[dbuf.at[pl.ds(g * GB, GB)]],
                                add=True)
                return 0

            lax.fori_loop(0, NGC, group_body, 0)
            return 0

        lax.fori_loop(0, NCH1, chunk_body, 0)
        plsc.subcore_barrier()
        wb = sid * SPT
        pltpu.sync_copy(acc.at[pl.ds(wb, SPT)],
                        out1.at[hglob].at[pl.ds(wb, SPT)])
        plsc.subcore_barrier()


def _stage_b(h1h, s_h, d_h, w_h):
    mesh = plsc.VectorSubcoreMesh(core_axis_name="c", subcore_axis_name="s",
                                  num_cores=NC, num_subcores=NS)
    f = pl.kernel(
        _stage_b_body,
        out_type=jax.ShapeDtypeStruct((HEADS, ACC_R, C1), jnp.float32),
        mesh=mesh,
        compiler_params=pltpu.CompilerParams(needs_layout_passes=False,
                                             use_tc_tiling_on_sc=False),
        scratch_types=[
            pltpu.VMEM_SHARED((ACC_R, C1), jnp.float32),
            pltpu.VMEM((CH,), jnp.int32),
            pltpu.VMEM((CH,), jnp.int32),
            pltpu.VMEM((CH,), jnp.float32),
            pltpu.VMEM((GB,), jnp.int32),
            pltpu.VMEM((GB, HID), jnp.float32),
            pltpu.VMEM((GB, C1), jnp.float32),
            pltpu.VMEM((8, C1), jnp.float32),
            pltpu.SemaphoreType.DMA,
        ],
    )
    return f(h1h, s_h, d_h, w_h)


def _stage_c_body(o1_ref, b1_ref, w2_ref, as2w_ref, ad2w_ref,
                  h2_ref, as2_ref, ad2_ref):
    h2 = jnp.zeros((BR, OUT_CH), jnp.float32)
    for hh in range(HEADS):
        m = o1_ref[hh, :, 0:HID]
        dn = o1_ref[hh, :, HID:HID + 1]
        a = m / (dn + _EPS) + b1_ref[0:1, hh * HID:(hh + 1) * HID]
        act = jnp.where(a > 0, a, jnp.exp(a) - 1.0)
        h2 = h2 + lax.dot_general(
            act, w2_ref[pl.ds(hh * HID, HID), :], (((1,), (0,)), ((), ())),
            preferred_element_type=jnp.float32)
    h2_ref[:, 0:OUT_CH] = h2
    h2_ref[:, OUT_CH:IN_CH] = jnp.zeros((BR, IN_CH - OUT_CH), jnp.float32)
    as2_ref[0] = lax.dot_general(as2w_ref[...], h2, (((1,), (1,)), ((), ())),
                                 preferred_element_type=jnp.float32)
    ad2_ref[0] = lax.dot_general(ad2w_ref[...], h2, (((1,), (1,)), ((), ())),
                                 preferred_element_type=jnp.float32)


def _stage_c(out1, b1, W2, att_src2, att_dst2):
    return pl.pallas_call(
        _stage_c_body,
        grid=(NB,),
        in_specs=[
            pl.BlockSpec((HEADS, BR, C1), lambda i: (0, i, 0)),
            pl.BlockSpec((1, HEADS * HID), lambda i: (0, 0)),
            pl.BlockSpec((HEADS * HID, OUT_CH), lambda i: (0, 0)),
            pl.BlockSpec((1, OUT_CH), lambda i: (0, 0)),
            pl.BlockSpec((1, OUT_CH), lambda i: (0, 0)),
        ],
        out_specs=[
            pl.BlockSpec((BR, IN_CH), lambda i: (i, 0)),
            pl.BlockSpec((1, 1, BR), lambda i: (i, 0, 0)),
            pl.BlockSpec((1, 1, BR), lambda i: (i, 0, 0)),
        ],
        out_shape=[
            jax.ShapeDtypeStruct((N, IN_CH), jnp.float32),
            jax.ShapeDtypeStruct((NB, 1, BR), jnp.float32),
            jax.ShapeDtypeStruct((NB, 1, BR), jnp.float32),
        ],
    )(out1, b1, W2, att_src2, att_dst2)


def _stage_d_body(h2p, as2, ad2, s_h, d_h, out2,
                  acc, asv, adv, sbuf, dbuf, wbuf, rows, scat, zbuf, sem):
    c = lax.axis_index("c")
    sid = lax.axis_index("s")
    wid = c * NS + sid
    iot = lax.iota(jnp.int32, L)
    ebase = wid * TPS2

    pltpu.sync_copy(s_h.at[pl.ds(ebase, TPS2)], sbuf)
    pltpu.sync_copy(d_h.at[pl.ds(ebase, TPS2)], dbuf)
    pltpu.sync_copy(as2.at[pl.ds(0, N)], asv.at[pl.ds(0, N)])
    pltpu.sync_copy(ad2.at[pl.ds(0, N)], adv.at[pl.ds(0, N)])
    _zero_acc(acc, zbuf, sid, C2)
    plsc.subcore_barrier()

    def group_body(g, _):
        for q in range(GBD // L):
            s16 = sbuf[pl.ds(g * GBD + q * L, L)]
            d16 = dbuf[pl.ds(g * GBD + q * L, L)]
            t = plsc.load_gather(asv, [s16]) + plsc.load_gather(adv, [d16])
            t = jnp.maximum(t, 0.2 * t)
            wbuf[pl.ds(q * L, L)] = jnp.exp(t)
        pltpu.async_copy(h2p.at[sbuf.at[pl.ds(g * GBD, GBD)]], rows,
                         sem).wait()

        def scale_body(j, _):
            wsp = plsc.load_gather(wbuf, [jnp.full((L,), 0, jnp.int32) + j])
            for k in range(OUT_CH // L):
                scat[j, pl.ds(k * L, L)] = rows[j, pl.ds(k * L, L)] * wsp
            scat[j, pl.ds(OUT_CH, L)] = jnp.where(iot == 0, wsp, 0.0)
            return 0

        lax.fori_loop(0, GBD, scale_body, 0)
        pltpu.sync_copy(scat, acc.at[dbuf.at[pl.ds(g * GBD, GBD)]], add=True)
        return 0

    lax.fori_loop(0, NGD, group_body, 0)
    plsc.subcore_barrier()
    wb = sid * SPT
    pltpu.sync_copy(acc.at[pl.ds(wb, SPT)], out2.at[c].at[pl.ds(wb, SPT)])


def _stage_d(h2p, as2, ad2, s_h, d_h):
    mesh = plsc.VectorSubcoreMesh(core_axis_name="c", subcore_axis_name="s",
                                  num_cores=NC, num_subcores=NS)
    f = pl.kernel(
        _stage_d_body,
        out_type=jax.ShapeDtypeStruct((NC, ACC_R, C2), jnp.float32),
        mesh=mesh,
        compiler_params=pltpu.CompilerParams(needs_layout_passes=False,
                                             use_tc_tiling_on_sc=False),
        scratch_types=[
            pltpu.VMEM_SHARED((ACC_R, C2), jnp.float32),
            pltpu.VMEM((N + L,), jnp.float32),
            pltpu.VMEM((N + L,), jnp.float32),
            pltpu.VMEM((TPS2,), jnp.int32),
            pltpu.VMEM((TPS2,), jnp.int32),
            pltpu.VMEM((GBD,), jnp.float32),
            pltpu.VMEM((GBD, IN_CH), jnp.float32),
            pltpu.VMEM((GBD, C2), jnp.float32),
            pltpu.VMEM((8, C2), jnp.float32),
            pltpu.SemaphoreType.DMA,
        ],
    )
    return f(h2p, as2, ad2, s_h, d_h)


def _stage_e_body(o2_ref, b2_ref, out_ref):
    m = o2_ref[0, :, 0:OUT_CH] + o2_ref[1, :, 0:OUT_CH]
    dn = o2_ref[0, :, OUT_CH:OUT_CH + 1] + o2_ref[1, :, OUT_CH:OUT_CH + 1]
    o = m / (dn + _EPS) + b2_ref[...]
    mx = jnp.max(o, axis=1, keepdims=True)
    e = jnp.exp(o - mx)
    s = jnp.sum(e, axis=1, keepdims=True)
    out_ref[...] = (o - mx) - jnp.log(s)


def _stage_e(out2, b2):
    return pl.pallas_call(
        _stage_e_body,
        grid=(NB,),
        in_specs=[
            pl.BlockSpec((NC, BR, C2), lambda i: (0, i, 0)),
            pl.BlockSpec((1, OUT_CH), lambda i: (0, 0)),
        ],
        out_specs=pl.BlockSpec((BR, OUT_CH), lambda i: (i, 0)),
        out_shape=jax.ShapeDtypeStruct((N, OUT_CH), jnp.float32),
    )(out2, b2)


def kernel(x, edge_index, W1, att_src1, att_dst1, b1, W2, att_src2, att_dst2, b2):
    src = edge_index[0]
    dst = edge_index[1]
    loop = jnp.arange(N, dtype=jnp.int32)
    pad = EPAD - E1
    s = jnp.concatenate([src, loop, jnp.zeros((pad,), jnp.int32)])
    d = jnp.concatenate([dst, loop, jnp.full((pad,), N, jnp.int32)])

    h1h, ast, adt = _stage_a(x, W1, att_src1, att_dst1)
    h1h = (h1h.reshape(NB, HEADS, BR, HID)
           .transpose(1, 0, 2, 3).reshape(HEADS * N, HID))
    ast = ast.transpose(1, 0, 2).reshape(HEADS * N)
    adt = adt.transpose(1, 0, 2).reshape(HEADS * N)
    w_h = _stage_b0(ast, adt, s, d)
    out1 = _stage_b(h1h, s, d, w_h)
    h2p, as2, ad2 = _stage_c(out1, b1.reshape(1, -1), W2, att_src2, att_dst2)
    out2 = _stage_d(h2p, as2.reshape(N), ad2.reshape(N), s, d)
    return _stage_e(out2, b2.reshape(1, -1))


# speedup vs baseline: 12.4896x; 1.0817x over previous
import jax
import jax.numpy as jnp
from jax import lax
from jax.experimental import pallas as pl
from jax.experimental.pallas import tpu as pltpu
from jax.experimental.pallas import tpu_sc as plsc

N = 10000
E_RAW = 320000
E1 = E_RAW + N
IN_CH = 128
HID = 128
HEADS = 4
OUT_CH = 64

NC = 2
NS = 16
L = 16

EPAD = 331776
TPS1 = EPAD // NS
TPS2 = EPAD // (NC * NS)

CH = 1152
NCH1 = TPS1 // CH
CH0 = 576
NCH0 = TPS2 // CH0
GB = 128
NGC = CH // GB
GBD = 128
NGD = TPS2 // GBD

ACC_R = 10112
SPT = ACC_R // NS
C1 = 144
C2 = 80

NB = 25
BR = N // NB

_EPS = 1e-30


def _stage_a_body(x_ref, w1_ref, asrc_ref, adst_ref, h1h_ref, as_ref, ad_ref):
    xb = x_ref[...]
    h = lax.dot_general(xb, w1_ref[...], (((1,), (0,)), ((), ())),
                        preferred_element_type=jnp.float32)
    for hh in range(HEADS):
        seg = h[:, hh * HID:(hh + 1) * HID]
        h1h_ref[pl.ds(hh * BR, BR), :] = seg
        as_ref[0, pl.ds(hh, 1), :] = lax.dot_general(
            asrc_ref[pl.ds(hh, 1), :], seg, (((1,), (1,)), ((), ())),
            preferred_element_type=jnp.float32)
        ad_ref[0, pl.ds(hh, 1), :] = lax.dot_general(
            adst_ref[pl.ds(hh, 1), :], seg, (((1,), (1,)), ((), ())),
            preferred_element_type=jnp.float32)


def _stage_a(x, W1, att_src1, att_dst1):
    return pl.pallas_call(
        _stage_a_body,
        grid=(NB,),
        in_specs=[
            pl.BlockSpec((BR, IN_CH), lambda i: (i, 0)),
            pl.BlockSpec((IN_CH, HEADS * HID), lambda i: (0, 0)),
            pl.BlockSpec((HEADS, HID), lambda i: (0, 0)),
            pl.BlockSpec((HEADS, HID), lambda i: (0, 0)),
        ],
        out_specs=[
            pl.BlockSpec((HEADS * BR, HID), lambda i: (i, 0)),
            pl.BlockSpec((1, HEADS, BR), lambda i: (i, 0, 0)),
            pl.BlockSpec((1, HEADS, BR), lambda i: (i, 0, 0)),
        ],
        out_shape=[
            jax.ShapeDtypeStruct((NB * HEADS * BR, HID), jnp.float32),
            jax.ShapeDtypeStruct((NB, HEADS, BR), jnp.float32),
            jax.ShapeDtypeStruct((NB, HEADS, BR), jnp.float32),
        ],
    )(x, W1, att_src1, att_dst1)


def _stage_b0_body(ast, adt, s_h, d_h, wout, asv, adv, sbuf, dbuf, w4):
    c = lax.axis_index("c")
    sid = lax.axis_index("s")
    wid = c * NS + sid
    pltpu.sync_copy(ast.at[pl.ds(0, HEADS * N)], asv.at[pl.ds(0, HEADS * N)])
    pltpu.sync_copy(adt.at[pl.ds(0, HEADS * N)], adv.at[pl.ds(0, HEADS * N)])

    def chunk_body(ch, _):
        base = wid * TPS2 + ch * CH0
        pltpu.sync_copy(s_h.at[pl.ds(base, CH0)], sbuf)
        pltpu.sync_copy(d_h.at[pl.ds(base, CH0)], dbuf)

        def q_body(q, _):
            s16 = sbuf[pl.ds(q * L, L)]
            d16 = dbuf[pl.ds(q * L, L)]
            for hh in range(HEADS):
                off = jnp.full((L,), hh * N, jnp.int32)
                t = (plsc.load_gather(asv, [s16 + off])
                     + plsc.load_gather(adv, [d16 + off]))
                t = jnp.maximum(t, 0.2 * t)
                w4[hh, pl.ds(q * L, L)] = jnp.exp(t)
            return 0

        lax.fori_loop(0, CH0 // L, q_body, 0)
        for hh in range(HEADS):
            pltpu.sync_copy(w4.at[hh], wout.at[pl.ds(hh * EPAD + base, CH0)])
        return 0

    lax.fori_loop(0, NCH0, chunk_body, 0)


def _stage_b0(ast, adt, s_h, d_h):
    mesh = plsc.VectorSubcoreMesh(core_axis_name="c", subcore_axis_name="s",
                                  num_cores=NC, num_subcores=NS)
    f = pl.kernel(
        _stage_b0_body,
        out_type=jax.ShapeDtypeStruct((HEADS * EPAD,), jnp.float32),
        mesh=mesh,
        compiler_params=pltpu.CompilerParams(needs_layout_passes=False,
                                             use_tc_tiling_on_sc=False),
        scratch_types=[
            pltpu.VMEM((HEADS * N + L,), jnp.float32),
            pltpu.VMEM((HEADS * N + L,), jnp.float32),
            pltpu.VMEM((CH0,), jnp.int32),
            pltpu.VMEM((CH0,), jnp.int32),
            pltpu.VMEM((HEADS, CH0), jnp.float32),
        ],
    )
    return f(ast, adt, s_h, d_h)


def _zero_acc(acc, zbuf, sid, cols):
    zv = jnp.zeros((L,), jnp.float32)
    for r in range(8):
        for k in range(cols // L):
            zbuf[r, pl.ds(k * L, L)] = zv
    base = sid * SPT
    for r in range(SPT // 8):
        pltpu.sync_copy(zbuf, acc.at[pl.ds(base + r * 8, 8)])


def _stage_b_body(h1h, s_h, d_h, w_h, out1,
                  acc, sbuf, dbuf, wbuf, gidx, rows, scat, zbuf, sem):
    c = lax.axis_index("c")
    sid = lax.axis_index("s")
    iot = lax.iota(jnp.int32, L)

    for hp in range(2):
        hglob = 2 * c + hp
        _zero_acc(acc, zbuf, sid, C1)
        plsc.subcore_barrier()

        def chunk_body(ch, _):
            base = sid * TPS1 + ch * CH
            pltpu.sync_copy(s_h.at[pl.ds(base, CH)], sbuf)
            pltpu.sync_copy(d_h.at[pl.ds(base, CH)], dbuf)
            pltpu.sync_copy(w_h.at[pl.ds(hglob * EPAD + base, CH)], wbuf)
            hoff = hglob * N

            def group_body(g, _):
                for q in range(GB // L):
                    s16 = sbuf[pl.ds(g * GB + q * L, L)]
                    gidx[pl.ds(q * L, L)] = s16 + hoff
                pltpu.async_copy(h1h.at[gidx], rows, sem).wait()

                def scale_body(j, _):
                    wsp = plsc.load_gather(
                        wbuf, [jnp.full((L,), 0, jnp.int32) + (g * GB + j)])
                    for k in range(HID // L):
                        scat[j, pl.ds(k * L, L)] = rows[j, pl.ds(k * L, L)] * wsp
                    scat[j, pl.ds(HID, L)] = jnp.where(iot == 0, wsp, 0.0)
                    return 0

                lax.fori_loop(0, GB, scale_body, 0)
                pltpu.sync_copy(scat, acc.at[dbuf.at[pl.ds(g * GB, GB)]],
                                add=True)
                return 0

            lax.fori_loop(0, NGC, group_body, 0)
            return 0

        lax.fori_loop(0, NCH1, chunk_body, 0)
        plsc.subcore_barrier()
        wb = sid * SPT
        pltpu.sync_copy(acc.at[pl.ds(wb, SPT)],
                        out1.at[hglob].at[pl.ds(wb, SPT)])
        plsc.subcore_barrier()


def _stage_b(h1h, s_h, d_h, w_h):
    mesh = plsc.VectorSubcoreMesh(core_axis_name="c", subcore_axis_name="s",
                                  num_cores=NC, num_subcores=NS)
    f = pl.kernel(
        _stage_b_body,
        out_type=jax.ShapeDtypeStruct((HEADS, ACC_R, C1), jnp.float32),
        mesh=mesh,
        compiler_params=pltpu.CompilerParams(needs_layout_passes=False,
                                             use_tc_tiling_on_sc=False),
        scratch_types=[
            pltpu.VMEM_SHARED((ACC_R, C1), jnp.float32),
            pltpu.VMEM((CH,), jnp.int32),
            pltpu.VMEM((CH,), jnp.int32),
            pltpu.VMEM((CH,), jnp.float32),
            pltpu.VMEM((GB,), jnp.int32),
            pltpu.VMEM((GB, HID), jnp.float32),
            pltpu.VMEM((GB, C1), jnp.float32),
            pltpu.VMEM((8, C1), jnp.float32),
            pltpu.SemaphoreType.DMA,
        ],
    )
    return f(h1h, s_h, d_h, w_h)


def _stage_c_body(o1_ref, b1_ref, w2_ref, as2w_ref, ad2w_ref,
                  h2_ref, as2_ref, ad2_ref):
    h2 = jnp.zeros((BR, OUT_CH), jnp.float32)
    for hh in range(HEADS):
        m = o1_ref[hh, :, 0:HID]
        dn = o1_ref[hh, :, HID:HID + 1]
        a = m / (dn + _EPS) + b1_ref[0:1, hh * HID:(hh + 1) * HID]
        act = jnp.where(a > 0, a, jnp.exp(a) - 1.0)
        h2 = h2 + lax.dot_general(
            act, w2_ref[pl.ds(hh * HID, HID), :], (((1,), (0,)), ((), ())),
            preferred_element_type=jnp.float32)
    h2_ref[:, 0:OUT_CH] = h2
    h2_ref[:, OUT_CH:IN_CH] = jnp.zeros((BR, IN_CH - OUT_CH), jnp.float32)
    as2_ref[0] = lax.dot_general(as2w_ref[...], h2, (((1,), (1,)), ((), ())),
                                 preferred_element_type=jnp.float32)
    ad2_ref[0] = lax.dot_general(ad2w_ref[...], h2, (((1,), (1,)), ((), ())),
                                 preferred_element_type=jnp.float32)


def _stage_c(out1, b1, W2, att_src2, att_dst2):
    return pl.pallas_call(
        _stage_c_body,
        grid=(NB,),
        in_specs=[
            pl.BlockSpec((HEADS, BR, C1), lambda i: (0, i, 0)),
            pl.BlockSpec((1, HEADS * HID), lambda i: (0, 0)),
            pl.BlockSpec((HEADS * HID, OUT_CH), lambda i: (0, 0)),
            pl.BlockSpec((1, OUT_CH), lambda i: (0, 0)),
            pl.BlockSpec((1, OUT_CH), lambda i: (0, 0)),
        ],
        out_specs=[
            pl.BlockSpec((BR, IN_CH), lambda i: (i, 0)),
            pl.BlockSpec((1, 1, BR), lambda i: (i, 0, 0)),
            pl.BlockSpec((1, 1, BR), lambda i: (i, 0, 0)),
        ],
        out_shape=[
            jax.ShapeDtypeStruct((N, IN_CH), jnp.float32),
            jax.ShapeDtypeStruct((NB, 1, BR), jnp.float32),
            jax.ShapeDtypeStruct((NB, 1, BR), jnp.float32),
        ],
    )(out1, b1, W2, att_src2, att_dst2)


def _stage_d_body(h2p, as2, ad2, s_h, d_h, out2,
                  acc, asv, adv, sbuf, dbuf, wbuf, rows, scat, zbuf, sem):
    c = lax.axis_index("c")
    sid = lax.axis_index("s")
    wid = c * NS + sid
    iot = lax.iota(jnp.int32, L)
    ebase = wid * TPS2

    pltpu.sync_copy(s_h.at[pl.ds(ebase, TPS2)], sbuf)
    pltpu.sync_copy(d_h.at[pl.ds(ebase, TPS2)], dbuf)
    pltpu.sync_copy(as2.at[pl.ds(0, N)], asv.at[pl.ds(0, N)])
    pltpu.sync_copy(ad2.at[pl.ds(0, N)], adv.at[pl.ds(0, N)])
    _zero_acc(acc, zbuf, sid, C2)
    plsc.subcore_barrier()

    def group_body(g, _):
        for q in range(GBD // L):
            s16 = sbuf[pl.ds(g * GBD + q * L, L)]
            d16 = dbuf[pl.ds(g * GBD + q * L, L)]
            t = plsc.load_gather(asv, [s16]) + plsc.load_gather(adv, [d16])
            t = jnp.maximum(t, 0.2 * t)
            wbuf[pl.ds(q * L, L)] = jnp.exp(t)
        pltpu.async_copy(h2p.at[sbuf.at[pl.ds(g * GBD, GBD)]], rows,
                         sem).wait()

        def scale_body(j, _):
            wsp = plsc.load_gather(wbuf, [jnp.full((L,), 0, jnp.int32) + j])
            for k in range(OUT_CH // L):
                scat[j, pl.ds(k * L, L)] = rows[j, pl.ds(k * L, L)] * wsp
            scat[j, pl.ds(OUT_CH, L)] = jnp.where(iot == 0, wsp, 0.0)
            return 0

        lax.fori_loop(0, GBD, scale_body, 0)
        pltpu.sync_copy(scat, acc.at[dbuf.at[pl.ds(g * GBD, GBD)]], add=True)
        return 0

    lax.fori_loop(0, NGD, group_body, 0)
    plsc.subcore_barrier()
    wb = sid * SPT
    pltpu.sync_copy(acc.at[pl.ds(wb, SPT)], out2.at[c].at[pl.ds(wb, SPT)])


def _stage_d(h2p, as2, ad2, s_h, d_h):
    mesh = plsc.VectorSubcoreMesh(core_axis_name="c", subcore_axis_name="s",
                                  num_cores=NC, num_subcores=NS)
    f = pl.kernel(
        _stage_d_body,
        out_type=jax.ShapeDtypeStruct((NC, ACC_R, C2), jnp.float32),
        mesh=mesh,
        compiler_params=pltpu.CompilerParams(needs_layout_passes=False,
                                             use_tc_tiling_on_sc=False),
        scratch_types=[
            pltpu.VMEM_SHARED((ACC_R, C2), jnp.float32),
            pltpu.VMEM((N + L,), jnp.float32),
            pltpu.VMEM((N + L,), jnp.float32),
            pltpu.VMEM((TPS2,), jnp.int32),
            pltpu.VMEM((TPS2,), jnp.int32),
            pltpu.VMEM((GBD,), jnp.float32),
            pltpu.VMEM((GBD, IN_CH), jnp.float32),
            pltpu.VMEM((GBD, C2), jnp.float32),
            pltpu.VMEM((8, C2), jnp.float32),
            pltpu.SemaphoreType.DMA,
        ],
    )
    return f(h2p, as2, ad2, s_h, d_h)


def _stage_e_body(o2_ref, b2_ref, out_ref):
    m = o2_ref[0, :, 0:OUT_CH] + o2_ref[1, :, 0:OUT_CH]
    dn = o2_ref[0, :, OUT_CH:OUT_CH + 1] + o2_ref[1, :, OUT_CH:OUT_CH + 1]
    o = m / (dn + _EPS) + b2_ref[...]
    mx = jnp.max(o, axis=1, keepdims=True)
    e = jnp.exp(o - mx)
    s = jnp.sum(e, axis=1, keepdims=True)
    out_ref[...] = (o - mx) - jnp.log(s)


def _stage_e(out2, b2):
    return pl.pallas_call(
        _stage_e_body,
        grid=(NB,),
        in_specs=[
            pl.BlockSpec((NC, BR, C2), lambda i: (0, i, 0)),
            pl.BlockSpec((1, OUT_CH), lambda i: (0, 0)),
        ],
        out_specs=pl.BlockSpec((BR, OUT_CH), lambda i: (i, 0)),
        out_shape=jax.ShapeDtypeStruct((N, OUT_CH), jnp.float32),
    )(out2, b2)


def kernel(x, edge_index, W1, att_src1, att_dst1, b1, W2, att_src2, att_dst2, b2):
    src = edge_index[0]
    dst = edge_index[1]
    loop = jnp.arange(N, dtype=jnp.int32)
    pad = EPAD - E1
    s = jnp.concatenate([src, loop, jnp.zeros((pad,), jnp.int32)])
    d = jnp.concatenate([dst, loop, jnp.full((pad,), N, jnp.int32)])

    h1h, ast, adt = _stage_a(x, W1, att_src1, att_dst1)
    h1h = (h1h.reshape(NB, HEADS, BR, HID)
           .transpose(1, 0, 2, 3).reshape(HEADS * N, HID))
    ast = ast.transpose(1, 0, 2).reshape(HEADS * N)
    adt = adt.transpose(1, 0, 2).reshape(HEADS * N)
    w_h = _stage_b0(ast, adt, s, d)
    out1 = _stage_b(h1h, s, d, w_h)
    h2p, as2, ad2 = _stage_c(out1, b1.reshape(1, -1), W2, att_src2, att_dst2)
    out2 = _stage_d(h2p, as2.reshape(N), ad2.reshape(N), s, d)
    return _stage_e(out2, b2.reshape(1, -1))


# speedup vs baseline: 14.5156x; 1.1622x over previous
import jax
import jax.numpy as jnp
from jax import lax
from jax.experimental import pallas as pl
from jax.experimental.pallas import tpu as pltpu
from jax.experimental.pallas import tpu_sc as plsc

N = 10000
E_RAW = 320000
E1 = E_RAW + N
IN_CH = 128
HID = 128
HEADS = 4
OUT_CH = 64

NC = 2
NS = 16
L = 16

EPAD = 331776
TPS1 = EPAD // NS
TPS2 = EPAD // (NC * NS)

CH = 1152
NCH1 = TPS1 // CH
CH0 = 576
NCH0 = TPS2 // CH0
GB = 64
NGC = CH // GB
NPAIR = NGC // 2
GBD = 128
NGD = TPS2 // GBD

ACC_R = 10112
SPT = ACC_R // NS
C1 = 144
C2 = 80

NB = 25
BR = N // NB

_EPS = 1e-30


def _stage_a_body(x_ref, w1_ref, asrc_ref, adst_ref, h1h_ref, as_ref, ad_ref):
    xb = x_ref[...]
    h = lax.dot_general(xb, w1_ref[...], (((1,), (0,)), ((), ())),
                        preferred_element_type=jnp.float32)
    for hh in range(HEADS):
        seg = h[:, hh * HID:(hh + 1) * HID]
        h1h_ref[pl.ds(hh * BR, BR), :] = seg
        as_ref[0, pl.ds(hh, 1), :] = lax.dot_general(
            asrc_ref[pl.ds(hh, 1), :], seg, (((1,), (1,)), ((), ())),
            preferred_element_type=jnp.float32)
        ad_ref[0, pl.ds(hh, 1), :] = lax.dot_general(
            adst_ref[pl.ds(hh, 1), :], seg, (((1,), (1,)), ((), ())),
            preferred_element_type=jnp.float32)


def _stage_a(x, W1, att_src1, att_dst1):
    return pl.pallas_call(
        _stage_a_body,
        grid=(NB,),
        in_specs=[
            pl.BlockSpec((BR, IN_CH), lambda i: (i, 0)),
            pl.BlockSpec((IN_CH, HEADS * HID), lambda i: (0, 0)),
            pl.BlockSpec((HEADS, HID), lambda i: (0, 0)),
            pl.BlockSpec((HEADS, HID), lambda i: (0, 0)),
        ],
        out_specs=[
            pl.BlockSpec((HEADS * BR, HID), lambda i: (i, 0)),
            pl.BlockSpec((1, HEADS, BR), lambda i: (i, 0, 0)),
            pl.BlockSpec((1, HEADS, BR), lambda i: (i, 0, 0)),
        ],
        out_shape=[
            jax.ShapeDtypeStruct((NB * HEADS * BR, HID), jnp.float32),
            jax.ShapeDtypeStruct((NB, HEADS, BR), jnp.float32),
            jax.ShapeDtypeStruct((NB, HEADS, BR), jnp.float32),
        ],
    )(x, W1, att_src1, att_dst1)


def _stage_b0_body(ast, adt, s_h, d_h, wout, asv, adv, sbuf, dbuf, w4):
    c = lax.axis_index("c")
    sid = lax.axis_index("s")
    wid = c * NS + sid
    pltpu.sync_copy(ast.at[pl.ds(0, HEADS * N)], asv.at[pl.ds(0, HEADS * N)])
    pltpu.sync_copy(adt.at[pl.ds(0, HEADS * N)], adv.at[pl.ds(0, HEADS * N)])

    def chunk_body(ch, _):
        base = wid * TPS2 + ch * CH0
        pltpu.sync_copy(s_h.at[pl.ds(base, CH0)], sbuf)
        pltpu.sync_copy(d_h.at[pl.ds(base, CH0)], dbuf)

        def q_body(q, _):
            s16 = sbuf[pl.ds(q * L, L)]
            d16 = dbuf[pl.ds(q * L, L)]
            for hh in range(HEADS):
                off = jnp.full((L,), hh * N, jnp.int32)
                t = (plsc.load_gather(asv, [s16 + off])
                     + plsc.load_gather(adv, [d16 + off]))
                t = jnp.maximum(t, 0.2 * t)
                w4[hh, pl.ds(q * L, L)] = jnp.exp(t)
            return 0

        lax.fori_loop(0, CH0 // L, q_body, 0)
        for hh in range(HEADS):
            pltpu.sync_copy(w4.at[hh], wout.at[pl.ds(hh * EPAD + base, CH0)])
        return 0

    lax.fori_loop(0, NCH0, chunk_body, 0)


def _stage_b0(ast, adt, s_h, d_h):
    mesh = plsc.VectorSubcoreMesh(core_axis_name="c", subcore_axis_name="s",
                                  num_cores=NC, num_subcores=NS)
    f = pl.kernel(
        _stage_b0_body,
        out_type=jax.ShapeDtypeStruct((HEADS * EPAD,), jnp.float32),
        mesh=mesh,
        compiler_params=pltpu.CompilerParams(needs_layout_passes=False,
                                             use_tc_tiling_on_sc=False),
        scratch_types=[
            pltpu.VMEM((HEADS * N + L,), jnp.float32),
            pltpu.VMEM((HEADS * N + L,), jnp.float32),
            pltpu.VMEM((CH0,), jnp.int32),
            pltpu.VMEM((CH0,), jnp.int32),
            pltpu.VMEM((HEADS, CH0), jnp.float32),
        ],
    )
    return f(ast, adt, s_h, d_h)


def _zero_acc(acc, zbuf, sid, cols):
    zv = jnp.zeros((L,), jnp.float32)
    for r in range(8):
        for k in range(cols // L):
            zbuf[r, pl.ds(k * L, L)] = zv
    base = sid * SPT
    for r in range(SPT // 8):
        pltpu.sync_copy(zbuf, acc.at[pl.ds(base + r * 8, 8)])


def _stage_b_body(h1h, s_h, d_h, w_h, out1,
                  acc, sbuf, dbuf, wbuf, gidx0, gidx1, rows0, rows1, scat,
                  zbuf, sem0, sem1):
    c = lax.axis_index("c")
    sid = lax.axis_index("s")
    iot = lax.iota(jnp.int32, L)

    for hp in range(2):
        hglob = 2 * c + hp
        hoff = hglob * N
        _zero_acc(acc, zbuf, sid, C1)
        plsc.subcore_barrier()

        def chunk_body(ch, _):
            base = sid * TPS1 + ch * CH
            pltpu.sync_copy(s_h.at[pl.ds(base, CH)], sbuf)
            pltpu.sync_copy(d_h.at[pl.ds(base, CH)], dbuf)
            pltpu.sync_copy(w_h.at[pl.ds(hglob * EPAD + base, CH)], wbuf)

            def fire(g, gidx, rows, sem):
                for q in range(GB // L):
                    s16 = sbuf[pl.ds(g * GB + q * L, L)]
                    gidx[pl.ds(q * L, L)] = s16 + hoff
                pltpu.async_copy(h1h.at[gidx], rows, sem)

            def drain(g, gidx, rows, sem):
                pltpu.make_async_copy(h1h.at[gidx], rows, sem).wait()

                def scale_body(j, _):
                    wsp = plsc.load_gather(
                        wbuf, [jnp.full((L,), 0, jnp.int32) + (g * GB + j)])
                    for k in range(HID // L):
                        scat[j, pl.ds(k * L, L)] = rows[j, pl.ds(k * L, L)] * wsp
                    scat[j, pl.ds(HID, L)] = jnp.where(iot == 0, wsp, 0.0)
                    return 0

                lax.fori_loop(0, GB, scale_body, 0)
                pltpu.sync_copy(scat, acc.at[dbuf.at[pl.ds(g * GB, GB)]],
                                add=True)

            fire(0, gidx0, rows0, sem0)

            def pair_body(pr, _):
                g = pr * 2
                fire(g + 1, gidx1, rows1, sem1)
                drain(g, gidx0, rows0, sem0)
                fire(g + 2, gidx0, rows0, sem0)
                drain(g + 1, gidx1, rows1, sem1)
                return 0

            lax.fori_loop(0, NPAIR - 1, pair_body, 0)
            gl = (NPAIR - 1) * 2
            fire(gl + 1, gidx1, rows1, sem1)
            drain(gl, gidx0, rows0, sem0)
            drain(gl + 1, gidx1, rows1, sem1)
            return 0

        lax.fori_loop(0, NCH1, chunk_body, 0)
        plsc.subcore_barrier()
        wb = sid * SPT
        pltpu.sync_copy(acc.at[pl.ds(wb, SPT)],
                        out1.at[hglob].at[pl.ds(wb, SPT)])
        plsc.subcore_barrier()


def _stage_b(h1h, s_h, d_h, w_h):
    mesh = plsc.VectorSubcoreMesh(core_axis_name="c", subcore_axis_name="s",
                                  num_cores=NC, num_subcores=NS)
    f = pl.kernel(
        _stage_b_body,
        out_type=jax.ShapeDtypeStruct((HEADS, ACC_R, C1), jnp.float32),
        mesh=mesh,
        compiler_params=pltpu.CompilerParams(needs_layout_passes=False,
                                             use_tc_tiling_on_sc=False),
        scratch_types=[
            pltpu.VMEM_SHARED((ACC_R, C1), jnp.float32),
            pltpu.VMEM((CH,), jnp.int32),
            pltpu.VMEM((CH,), jnp.int32),
            pltpu.VMEM((CH,), jnp.float32),
            pltpu.VMEM((GB,), jnp.int32),
            pltpu.VMEM((GB,), jnp.int32),
            pltpu.VMEM((GB, HID), jnp.float32),
            pltpu.VMEM((GB, HID), jnp.float32),
            pltpu.VMEM((GB, C1), jnp.float32),
            pltpu.VMEM((8, C1), jnp.float32),
            pltpu.SemaphoreType.DMA,
            pltpu.SemaphoreType.DMA,
        ],
    )
    return f(h1h, s_h, d_h, w_h)


def _stage_c_body(o1_ref, b1_ref, w2_ref, as2w_ref, ad2w_ref,
                  h2_ref, as2_ref, ad2_ref):
    h2 = jnp.zeros((BR, OUT_CH), jnp.float32)
    for hh in range(HEADS):
        m = o1_ref[hh, :, 0:HID]
        dn = o1_ref[hh, :, HID:HID + 1]
        a = m / (dn + _EPS) + b1_ref[0:1, hh * HID:(hh + 1) * HID]
        act = jnp.where(a > 0, a, jnp.exp(a) - 1.0)
        h2 = h2 + lax.dot_general(
            act, w2_ref[pl.ds(hh * HID, HID), :], (((1,), (0,)), ((), ())),
            preferred_element_type=jnp.float32)
    h2_ref[:, 0:OUT_CH] = h2
    h2_ref[:, OUT_CH:IN_CH] = jnp.zeros((BR, IN_CH - OUT_CH), jnp.float32)
    as2_ref[0] = lax.dot_general(as2w_ref[...], h2, (((1,), (1,)), ((), ())),
                                 preferred_element_type=jnp.float32)
    ad2_ref[0] = lax.dot_general(ad2w_ref[...], h2, (((1,), (1,)), ((), ())),
                                 preferred_element_type=jnp.float32)


def _stage_c(out1, b1, W2, att_src2, att_dst2):
    return pl.pallas_call(
        _stage_c_body,
        grid=(NB,),
        in_specs=[
            pl.BlockSpec((HEADS, BR, C1), lambda i: (0, i, 0)),
            pl.BlockSpec((1, HEADS * HID), lambda i: (0, 0)),
            pl.BlockSpec((HEADS * HID, OUT_CH), lambda i: (0, 0)),
            pl.BlockSpec((1, OUT_CH), lambda i: (0, 0)),
            pl.BlockSpec((1, OUT_CH), lambda i: (0, 0)),
        ],
        out_specs=[
            pl.BlockSpec((BR, IN_CH), lambda i: (i, 0)),
            pl.BlockSpec((1, 1, BR), lambda i: (i, 0, 0)),
            pl.BlockSpec((1, 1, BR), lambda i: (i, 0, 0)),
        ],
        out_shape=[
            jax.ShapeDtypeStruct((N, IN_CH), jnp.float32),
            jax.ShapeDtypeStruct((NB, 1, BR), jnp.float32),
            jax.ShapeDtypeStruct((NB, 1, BR), jnp.float32),
        ],
    )(out1, b1, W2, att_src2, att_dst2)


def _stage_d_body(h2p, as2, ad2, s_h, d_h, out2,
                  acc, asv, adv, sbuf, dbuf, wbuf, rows, scat, zbuf, sem):
    c = lax.axis_index("c")
    sid = lax.axis_index("s")
    wid = c * NS + sid
    iot = lax.iota(jnp.int32, L)
    ebase = wid * TPS2

    pltpu.sync_copy(s_h.at[pl.ds(ebase, TPS2)], sbuf)
    pltpu.sync_copy(d_h.at[pl.ds(ebase, TPS2)], dbuf)
    pltpu.sync_copy(as2.at[pl.ds(0, N)], asv.at[pl.ds(0, N)])
    pltpu.sync_copy(ad2.at[pl.ds(0, N)], adv.at[pl.ds(0, N)])
    _zero_acc(acc, zbuf, sid, C2)
    plsc.subcore_barrier()

    def group_body(g, _):
        for q in range(GBD // L):
            s16 = sbuf[pl.ds(g * GBD + q * L, L)]
            d16 = dbuf[pl.ds(g * GBD + q * L, L)]
            t = plsc.load_gather(asv, [s16]) + plsc.load_gather(adv, [d16])
            t = jnp.maximum(t, 0.2 * t)
            wbuf[pl.ds(q * L, L)] = jnp.exp(t)
        pltpu.async_copy(h2p.at[sbuf.at[pl.ds(g * GBD, GBD)]], rows,
                         sem).wait()

        def scale_body(j, _):
            wsp = plsc.load_gather(wbuf, [jnp.full((L,), 0, jnp.int32) + j])
            for k in range(OUT_CH // L):
                scat[j, pl.ds(k * L, L)] = rows[j, pl.ds(k * L, L)] * wsp
            scat[j, pl.ds(OUT_CH, L)] = jnp.where(iot == 0, wsp, 0.0)
            return 0

        lax.fori_loop(0, GBD, scale_body, 0)
        pltpu.sync_copy(scat, acc.at[dbuf.at[pl.ds(g * GBD, GBD)]], add=True)
        return 0

    lax.fori_loop(0, NGD, group_body, 0)
    plsc.subcore_barrier()
    wb = sid * SPT
    pltpu.sync_copy(acc.at[pl.ds(wb, SPT)], out2.at[c].at[pl.ds(wb, SPT)])


def _stage_d(h2p, as2, ad2, s_h, d_h):
    mesh = plsc.VectorSubcoreMesh(core_axis_name="c", subcore_axis_name="s",
                                  num_cores=NC, num_subcores=NS)
    f = pl.kernel(
        _stage_d_body,
        out_type=jax.ShapeDtypeStruct((NC, ACC_R, C2), jnp.float32),
        mesh=mesh,
        compiler_params=pltpu.CompilerParams(needs_layout_passes=False,
                                             use_tc_tiling_on_sc=False),
        scratch_types=[
            pltpu.VMEM_SHARED((ACC_R, C2), jnp.float32),
            pltpu.VMEM((N + L,), jnp.float32),
            pltpu.VMEM((N + L,), jnp.float32),
            pltpu.VMEM((TPS2,), jnp.int32),
            pltpu.VMEM((TPS2,), jnp.int32),
            pltpu.VMEM((GBD,), jnp.float32),
            pltpu.VMEM((GBD, IN_CH), jnp.float32),
            pltpu.VMEM((GBD, C2), jnp.float32),
            pltpu.VMEM((8, C2), jnp.float32),
            pltpu.SemaphoreType.DMA,
        ],
    )
    return f(h2p, as2, ad2, s_h, d_h)


def _stage_e_body(o2_ref, b2_ref, out_ref):
    m = o2_ref[0, :, 0:OUT_CH] + o2_ref[1, :, 0:OUT_CH]
    dn = o2_ref[0, :, OUT_CH:OUT_CH + 1] + o2_ref[1, :, OUT_CH:OUT_CH + 1]
    o = m / (dn + _EPS) + b2_ref[...]
    mx = jnp.max(o, axis=1, keepdims=True)
    e = jnp.exp(o - mx)
    s = jnp.sum(e, axis=1, keepdims=True)
    out_ref[...] = (o - mx) - jnp.log(s)


def _stage_e(out2, b2):
    return pl.pallas_call(
        _stage_e_body,
        grid=(NB,),
        in_specs=[
            pl.BlockSpec((NC, BR, C2), lambda i: (0, i, 0)),
            pl.BlockSpec((1, OUT_CH), lambda i: (0, 0)),
        ],
        out_specs=pl.BlockSpec((BR, OUT_CH), lambda i: (i, 0)),
        out_shape=jax.ShapeDtypeStruct((N, OUT_CH), jnp.float32),
    )(out2, b2)


def kernel(x, edge_index, W1, att_src1, att_dst1, b1, W2, att_src2, att_dst2, b2):
    src = edge_index[0]
    dst = edge_index[1]
    loop = jnp.arange(N, dtype=jnp.int32)
    pad = EPAD - E1
    s = jnp.concatenate([src, loop, jnp.zeros((pad,), jnp.int32)])
    d = jnp.concatenate([dst, loop, jnp.full((pad,), N, jnp.int32)])

    h1h, ast, adt = _stage_a(x, W1, att_src1, att_dst1)
    h1h = (h1h.reshape(NB, HEADS, BR, HID)
           .transpose(1, 0, 2, 3).reshape(HEADS * N, HID))
    ast = ast.transpose(1, 0, 2).reshape(HEADS * N)
    adt = adt.transpose(1, 0, 2).reshape(HEADS * N)
    w_h = _stage_b0(ast, adt, s, d)
    out1 = _stage_b(h1h, s, d, w_h)
    h2p, as2, ad2 = _stage_c(out1, b1.reshape(1, -1), W2, att_src2, att_dst2)
    out2 = _stage_d(h2p, as2.reshape(N), ad2.reshape(N), s, d)
    return _stage_e(out2, b2.reshape(1, -1))


# speedup vs baseline: 15.2440x; 1.0502x over previous
import jax
import jax.numpy as jnp
from jax import lax
from jax.experimental import pallas as pl
from jax.experimental.pallas import tpu as pltpu
from jax.experimental.pallas import tpu_sc as plsc

N = 10000
E_RAW = 320000
E1 = E_RAW + N
IN_CH = 128
HID = 128
HEADS = 4
OUT_CH = 64

NC = 2
NS = 16
L = 16

EPAD = 331776
TPS1 = EPAD // NS
TPS2 = EPAD // (NC * NS)

CH = 1152
NCH1 = TPS1 // CH
CH0 = 576
NCH0 = TPS2 // CH0
GB = 64
NGC = CH // GB
NPAIR = NGC // 2
GBD = 96
NGD = TPS2 // GBD
NPAIRD = NGD // 2

ACC_R = 10112
SPT = ACC_R // NS
C1 = 144
C2 = 80

NB = 25
BR = N // NB

_EPS = 1e-30


def _stage_a_body(x_ref, w1_ref, asrc_ref, adst_ref, h1h_ref, as_ref, ad_ref):
    xb = x_ref[...]
    h = lax.dot_general(xb, w1_ref[...], (((1,), (0,)), ((), ())),
                        preferred_element_type=jnp.float32)
    for hh in range(HEADS):
        seg = h[:, hh * HID:(hh + 1) * HID]
        h1h_ref[pl.ds(hh * BR, BR), :] = seg
        as_ref[0, pl.ds(hh, 1), :] = lax.dot_general(
            asrc_ref[pl.ds(hh, 1), :], seg, (((1,), (1,)), ((), ())),
            preferred_element_type=jnp.float32)
        ad_ref[0, pl.ds(hh, 1), :] = lax.dot_general(
            adst_ref[pl.ds(hh, 1), :], seg, (((1,), (1,)), ((), ())),
            preferred_element_type=jnp.float32)


def _stage_a(x, W1, att_src1, att_dst1):
    return pl.pallas_call(
        _stage_a_body,
        grid=(NB,),
        in_specs=[
            pl.BlockSpec((BR, IN_CH), lambda i: (i, 0)),
            pl.BlockSpec((IN_CH, HEADS * HID), lambda i: (0, 0)),
            pl.BlockSpec((HEADS, HID), lambda i: (0, 0)),
            pl.BlockSpec((HEADS, HID), lambda i: (0, 0)),
        ],
        out_specs=[
            pl.BlockSpec((HEADS * BR, HID), lambda i: (i, 0)),
            pl.BlockSpec((1, HEADS, BR), lambda i: (i, 0, 0)),
            pl.BlockSpec((1, HEADS, BR), lambda i: (i, 0, 0)),
        ],
        out_shape=[
            jax.ShapeDtypeStruct((NB * HEADS * BR, HID), jnp.float32),
            jax.ShapeDtypeStruct((NB, HEADS, BR), jnp.float32),
            jax.ShapeDtypeStruct((NB, HEADS, BR), jnp.float32),
        ],
    )(x, W1, att_src1, att_dst1)


def _stage_b0_body(ast, adt, s_h, d_h, wout, asv, adv, sbuf, dbuf, w4):
    c = lax.axis_index("c")
    sid = lax.axis_index("s")
    wid = c * NS + sid
    pltpu.sync_copy(ast.at[pl.ds(0, HEADS * N)], asv.at[pl.ds(0, HEADS * N)])
    pltpu.sync_copy(adt.at[pl.ds(0, HEADS * N)], adv.at[pl.ds(0, HEADS * N)])

    def chunk_body(ch, _):
        base = wid * TPS2 + ch * CH0
        pltpu.sync_copy(s_h.at[pl.ds(base, CH0)], sbuf)
        pltpu.sync_copy(d_h.at[pl.ds(base, CH0)], dbuf)

        def q_body(q, _):
            s16 = sbuf[pl.ds(q * L, L)]
            d16 = dbuf[pl.ds(q * L, L)]
            for hh in range(HEADS):
                off = jnp.full((L,), hh * N, jnp.int32)
                t = (plsc.load_gather(asv, [s16 + off])
                     + plsc.load_gather(adv, [d16 + off]))
                t = jnp.maximum(t, 0.2 * t)
                w4[hh, pl.ds(q * L, L)] = jnp.exp(t)
            return 0

        lax.fori_loop(0, CH0 // L, q_body, 0)
        for hh in range(HEADS):
            pltpu.sync_copy(w4.at[hh], wout.at[pl.ds(hh * EPAD + base, CH0)])
        return 0

    lax.fori_loop(0, NCH0, chunk_body, 0)


def _stage_b0(ast, adt, s_h, d_h):
    mesh = plsc.VectorSubcoreMesh(core_axis_name="c", subcore_axis_name="s",
                                  num_cores=NC, num_subcores=NS)
    f = pl.kernel(
        _stage_b0_body,
        out_type=jax.ShapeDtypeStruct((HEADS * EPAD,), jnp.float32),
        mesh=mesh,
        compiler_params=pltpu.CompilerParams(needs_layout_passes=False,
                                             use_tc_tiling_on_sc=False),
        scratch_types=[
            pltpu.VMEM((HEADS * N + L,), jnp.float32),
            pltpu.VMEM((HEADS * N + L,), jnp.float32),
            pltpu.VMEM((CH0,), jnp.int32),
            pltpu.VMEM((CH0,), jnp.int32),
            pltpu.VMEM((HEADS, CH0), jnp.float32),
        ],
    )
    return f(ast, adt, s_h, d_h)


def _zero_acc(acc, zbuf, sid, cols):
    zv = jnp.zeros((L,), jnp.float32)
    for r in range(8):
        for k in range(cols // L):
            zbuf[r, pl.ds(k * L, L)] = zv
    base = sid * SPT
    for r in range(SPT // 8):
        pltpu.sync_copy(zbuf, acc.at[pl.ds(base + r * 8, 8)])


def _stage_b_body(h1h, s_h, d_h, w_h, out1,
                  acc, sbuf, dbuf, wbuf, gidx0, gidx1, rows0, rows1, scat,
                  zbuf, sem0, sem1):
    c = lax.axis_index("c")
    sid = lax.axis_index("s")
    iot = lax.iota(jnp.int32, L)

    for hp in range(2):
        hglob = 2 * c + hp
        hoff = hglob * N
        _zero_acc(acc, zbuf, sid, C1)
        plsc.subcore_barrier()

        def chunk_body(ch, _):
            base = sid * TPS1 + ch * CH
            pltpu.sync_copy(s_h.at[pl.ds(base, CH)], sbuf)
            pltpu.sync_copy(d_h.at[pl.ds(base, CH)], dbuf)
            pltpu.sync_copy(w_h.at[pl.ds(hglob * EPAD + base, CH)], wbuf)

            def fire(g, gidx, rows, sem):
                for q in range(GB // L):
                    s16 = sbuf[pl.ds(g * GB + q * L, L)]
                    gidx[pl.ds(q * L, L)] = s16 + hoff
                pltpu.async_copy(h1h.at[gidx], rows, sem)

            def drain(g, gidx, rows, sem):
                pltpu.make_async_copy(h1h.at[gidx], rows, sem).wait()

                def scale_body(j, _):
                    wsp = plsc.load_gather(
                        wbuf, [jnp.full((L,), 0, jnp.int32) + (g * GB + j)])
                    for k in range(HID // L):
                        scat[j, pl.ds(k * L, L)] = rows[j, pl.ds(k * L, L)] * wsp
                    scat[j, pl.ds(HID, L)] = jnp.where(iot == 0, wsp, 0.0)
                    return 0

                lax.fori_loop(0, GB, scale_body, 0)
                pltpu.sync_copy(scat, acc.at[dbuf.at[pl.ds(g * GB, GB)]],
                                add=True)

            fire(0, gidx0, rows0, sem0)

            def pair_body(pr, _):
                g = pr * 2
                fire(g + 1, gidx1, rows1, sem1)
                drain(g, gidx0, rows0, sem0)
                fire(g + 2, gidx0, rows0, sem0)
                drain(g + 1, gidx1, rows1, sem1)
                return 0

            lax.fori_loop(0, NPAIR - 1, pair_body, 0)
            gl = (NPAIR - 1) * 2
            fire(gl + 1, gidx1, rows1, sem1)
            drain(gl, gidx0, rows0, sem0)
            drain(gl + 1, gidx1, rows1, sem1)
            return 0

        lax.fori_loop(0, NCH1, chunk_body, 0)
        plsc.subcore_barrier()
        wb = sid * SPT
        pltpu.sync_copy(acc.at[pl.ds(wb, SPT)],
                        out1.at[hglob].at[pl.ds(wb, SPT)])
        plsc.subcore_barrier()


def _stage_b(h1h, s_h, d_h, w_h):
    mesh = plsc.VectorSubcoreMesh(core_axis_name="c", subcore_axis_name="s",
                                  num_cores=NC, num_subcores=NS)
    f = pl.kernel(
        _stage_b_body,
        out_type=jax.ShapeDtypeStruct((HEADS, ACC_R, C1), jnp.float32),
        mesh=mesh,
        compiler_params=pltpu.CompilerParams(needs_layout_passes=False,
                                             use_tc_tiling_on_sc=False),
        scratch_types=[
            pltpu.VMEM_SHARED((ACC_R, C1), jnp.float32),
            pltpu.VMEM((CH,), jnp.int32),
            pltpu.VMEM((CH,), jnp.int32),
            pltpu.VMEM((CH,), jnp.float32),
            pltpu.VMEM((GB,), jnp.int32),
            pltpu.VMEM((GB,), jnp.int32),
            pltpu.VMEM((GB, HID), jnp.float32),
            pltpu.VMEM((GB, HID), jnp.float32),
            pltpu.VMEM((GB, C1), jnp.float32),
            pltpu.VMEM((8, C1), jnp.float32),
            pltpu.SemaphoreType.DMA,
            pltpu.SemaphoreType.DMA,
        ],
    )
    return f(h1h, s_h, d_h, w_h)


def _stage_c_body(o1_ref, b1_ref, w2_ref, as2w_ref, ad2w_ref,
                  h2_ref, as2_ref, ad2_ref):
    h2 = jnp.zeros((BR, OUT_CH), jnp.float32)
    for hh in range(HEADS):
        m = o1_ref[hh, :, 0:HID]
        dn = o1_ref[hh, :, HID:HID + 1]
        a = m / (dn + _EPS) + b1_ref[0:1, hh * HID:(hh + 1) * HID]
        act = jnp.where(a > 0, a, jnp.exp(a) - 1.0)
        h2 = h2 + lax.dot_general(
            act, w2_ref[pl.ds(hh * HID, HID), :], (((1,), (0,)), ((), ())),
            preferred_element_type=jnp.float32)
    h2_ref[:, 0:OUT_CH] = h2
    h2_ref[:, OUT_CH:IN_CH] = jnp.zeros((BR, IN_CH - OUT_CH), jnp.float32)
    as2_ref[0] = lax.dot_general(as2w_ref[...], h2, (((1,), (1,)), ((), ())),
                                 preferred_element_type=jnp.float32)
    ad2_ref[0] = lax.dot_general(ad2w_ref[...], h2, (((1,), (1,)), ((), ())),
                                 preferred_element_type=jnp.float32)


def _stage_c(out1, b1, W2, att_src2, att_dst2):
    return pl.pallas_call(
        _stage_c_body,
        grid=(NB,),
        in_specs=[
            pl.BlockSpec((HEADS, BR, C1), lambda i: (0, i, 0)),
            pl.BlockSpec((1, HEADS * HID), lambda i: (0, 0)),
            pl.BlockSpec((HEADS * HID, OUT_CH), lambda i: (0, 0)),
            pl.BlockSpec((1, OUT_CH), lambda i: (0, 0)),
            pl.BlockSpec((1, OUT_CH), lambda i: (0, 0)),
        ],
        out_specs=[
            pl.BlockSpec((BR, IN_CH), lambda i: (i, 0)),
            pl.BlockSpec((1, 1, BR), lambda i: (i, 0, 0)),
            pl.BlockSpec((1, 1, BR), lambda i: (i, 0, 0)),
        ],
        out_shape=[
            jax.ShapeDtypeStruct((N, IN_CH), jnp.float32),
            jax.ShapeDtypeStruct((NB, 1, BR), jnp.float32),
            jax.ShapeDtypeStruct((NB, 1, BR), jnp.float32),
        ],
    )(out1, b1, W2, att_src2, att_dst2)


def _stage_d_body(h2p, as2, ad2, s_h, d_h, out2,
                  acc, asv, adv, sbuf, dbuf, wbuf, rows0, rows1, scat, zbuf,
                  sem0, sem1):
    c = lax.axis_index("c")
    sid = lax.axis_index("s")
    wid = c * NS + sid
    iot = lax.iota(jnp.int32, L)
    ebase = wid * TPS2

    pltpu.sync_copy(s_h.at[pl.ds(ebase, TPS2)], sbuf)
    pltpu.sync_copy(d_h.at[pl.ds(ebase, TPS2)], dbuf)
    pltpu.sync_copy(as2.at[pl.ds(0, N)], asv.at[pl.ds(0, N)])
    pltpu.sync_copy(ad2.at[pl.ds(0, N)], adv.at[pl.ds(0, N)])
    _zero_acc(acc, zbuf, sid, C2)
    plsc.subcore_barrier()

    def fire(g, rows, sem):
        pltpu.async_copy(h2p.at[sbuf.at[pl.ds(g * GBD, GBD)]], rows, sem)

    def drain(g, rows, sem):
        pltpu.make_async_copy(h2p.at[sbuf.at[pl.ds(g * GBD, GBD)]], rows,
                              sem).wait()
        for q in range(GBD // L):
            s16 = sbuf[pl.ds(g * GBD + q * L, L)]
            d16 = dbuf[pl.ds(g * GBD + q * L, L)]
            t = plsc.load_gather(asv, [s16]) + plsc.load_gather(adv, [d16])
            t = jnp.maximum(t, 0.2 * t)
            wbuf[pl.ds(q * L, L)] = jnp.exp(t)

        def scale_body(j, _):
            wsp = plsc.load_gather(wbuf, [jnp.full((L,), 0, jnp.int32) + j])
            for k in range(OUT_CH // L):
                scat[j, pl.ds(k * L, L)] = rows[j, pl.ds(k * L, L)] * wsp
            scat[j, pl.ds(OUT_CH, L)] = jnp.where(iot == 0, wsp, 0.0)
            return 0

        lax.fori_loop(0, GBD, scale_body, 0)
        pltpu.sync_copy(scat, acc.at[dbuf.at[pl.ds(g * GBD, GBD)]], add=True)

    fire(0, rows0, sem0)

    def pair_body(pr, _):
        g = pr * 2
        fire(g + 1, rows1, sem1)
        drain(g, rows0, sem0)
        fire(g + 2, rows0, sem0)
        drain(g + 1, rows1, sem1)
        return 0

    lax.fori_loop(0, NPAIRD - 1, pair_body, 0)
    gl = (NPAIRD - 1) * 2
    fire(gl + 1, rows1, sem1)
    drain(gl, rows0, sem0)
    drain(gl + 1, rows1, sem1)
    plsc.subcore_barrier()
    wb = sid * SPT
    pltpu.sync_copy(acc.at[pl.ds(wb, SPT)], out2.at[c].at[pl.ds(wb, SPT)])


def _stage_d(h2p, as2, ad2, s_h, d_h):
    mesh = plsc.VectorSubcoreMesh(core_axis_name="c", subcore_axis_name="s",
                                  num_cores=NC, num_subcores=NS)
    f = pl.kernel(
        _stage_d_body,
        out_type=jax.ShapeDtypeStruct((NC, ACC_R, C2), jnp.float32),
        mesh=mesh,
        compiler_params=pltpu.CompilerParams(needs_layout_passes=False,
                                             use_tc_tiling_on_sc=False),
        scratch_types=[
            pltpu.VMEM_SHARED((ACC_R, C2), jnp.float32),
            pltpu.VMEM((N + L,), jnp.float32),
            pltpu.VMEM((N + L,), jnp.float32),
            pltpu.VMEM((TPS2,), jnp.int32),
            pltpu.VMEM((TPS2,), jnp.int32),
            pltpu.VMEM((GBD,), jnp.float32),
            pltpu.VMEM((GBD, IN_CH), jnp.float32),
            pltpu.VMEM((GBD, IN_CH), jnp.float32),
            pltpu.VMEM((GBD, C2), jnp.float32),
            pltpu.VMEM((8, C2), jnp.float32),
            pltpu.SemaphoreType.DMA,
            pltpu.SemaphoreType.DMA,
        ],
    )
    return f(h2p, as2, ad2, s_h, d_h)


def _stage_e_body(o2_ref, b2_ref, out_ref):
    m = o2_ref[0, :, 0:OUT_CH] + o2_ref[1, :, 0:OUT_CH]
    dn = o2_ref[0, :, OUT_CH:OUT_CH + 1] + o2_ref[1, :, OUT_CH:OUT_CH + 1]
    o = m / (dn + _EPS) + b2_ref[...]
    mx = jnp.max(o, axis=1, keepdims=True)
    e = jnp.exp(o - mx)
    s = jnp.sum(e, axis=1, keepdims=True)
    out_ref[...] = (o - mx) - jnp.log(s)


def _stage_e(out2, b2):
    return pl.pallas_call(
        _stage_e_body,
        grid=(NB,),
        in_specs=[
            pl.BlockSpec((NC, BR, C2), lambda i: (0, i, 0)),
            pl.BlockSpec((1, OUT_CH), lambda i: (0, 0)),
        ],
        out_specs=pl.BlockSpec((BR, OUT_CH), lambda i: (i, 0)),
        out_shape=jax.ShapeDtypeStruct((N, OUT_CH), jnp.float32),
    )(out2, b2)


def kernel(x, edge_index, W1, att_src1, att_dst1, b1, W2, att_src2, att_dst2, b2):
    src = edge_index[0]
    dst = edge_index[1]
    loop = jnp.arange(N, dtype=jnp.int32)
    pad = EPAD - E1
    s = jnp.concatenate([src, loop, jnp.zeros((pad,), jnp.int32)])
    d = jnp.concatenate([dst, loop, jnp.full((pad,), N, jnp.int32)])

    h1h, ast, adt = _stage_a(x, W1, att_src1, att_dst1)
    h1h = (h1h.reshape(NB, HEADS, BR, HID)
           .transpose(1, 0, 2, 3).reshape(HEADS * N, HID))
    ast = ast.transpose(1, 0, 2).reshape(HEADS * N)
    adt = adt.transpose(1, 0, 2).reshape(HEADS * N)
    w_h = _stage_b0(ast, adt, s, d)
    out1 = _stage_b(h1h, s, d, w_h)
    h2p, as2, ad2 = _stage_c(out1, b1.reshape(1, -1), W2, att_src2, att_dst2)
    out2 = _stage_d(h2p, as2.reshape(N), ad2.reshape(N), s, d)
    return _stage_e(out2, b2.reshape(1, -1))


# speedup vs baseline: 30.5009x; 2.0008x over previous
import jax
import jax.numpy as jnp
from jax import lax
from jax.experimental import pallas as pl
from jax.experimental.pallas import tpu as pltpu
from jax.experimental.pallas import tpu_sc as plsc

N = 10000
E_RAW = 320000
E1 = E_RAW + N
IN_CH = 128
HID = 128
HEADS = 4
OUT_CH = 64

NC = 2
NS = 16
L = 16

EPAD = 331776
TPS1 = EPAD // NS
TPS2 = EPAD // (NC * NS)

CH = 1152
NCH1 = TPS1 // CH
CH0 = 576
NCH0 = TPS2 // CH0
GB = 64
NGC = CH // GB
NPAIR = NGC // 2
GBD = 96
NGD = TPS2 // GBD
NPAIRD = NGD // 2

ACC_R = 10112
SPT = ACC_R // NS
C1 = 144
C2 = 80

NB = 25
BR = N // NB

_EPS = 1e-30


def _stage_a_body(x_ref, w1_ref, asrc_ref, adst_ref, h1h_ref, as_ref, ad_ref):
    xb = x_ref[...]
    h = lax.dot_general(xb, w1_ref[...], (((1,), (0,)), ((), ())),
                        preferred_element_type=jnp.float32)
    for hh in range(HEADS):
        seg = h[:, hh * HID:(hh + 1) * HID]
        h1h_ref[pl.ds(hh * BR, BR), :] = seg
        as_ref[0, pl.ds(hh, 1), :] = lax.dot_general(
            asrc_ref[pl.ds(hh, 1), :], seg, (((1,), (1,)), ((), ())),
            preferred_element_type=jnp.float32)
        ad_ref[0, pl.ds(hh, 1), :] = lax.dot_general(
            adst_ref[pl.ds(hh, 1), :], seg, (((1,), (1,)), ((), ())),
            preferred_element_type=jnp.float32)


def _stage_a(x, W1, att_src1, att_dst1):
    return pl.pallas_call(
        _stage_a_body,
        grid=(NB,),
        in_specs=[
            pl.BlockSpec((BR, IN_CH), lambda i: (i, 0)),
            pl.BlockSpec((IN_CH, HEADS * HID), lambda i: (0, 0)),
            pl.BlockSpec((HEADS, HID), lambda i: (0, 0)),
            pl.BlockSpec((HEADS, HID), lambda i: (0, 0)),
        ],
        out_specs=[
            pl.BlockSpec((HEADS * BR, HID), lambda i: (i, 0)),
            pl.BlockSpec((1, HEADS, BR), lambda i: (i, 0, 0)),
            pl.BlockSpec((1, HEADS, BR), lambda i: (i, 0, 0)),
        ],
        out_shape=[
            jax.ShapeDtypeStruct((NB * HEADS * BR, HID), jnp.float32),
            jax.ShapeDtypeStruct((NB, HEADS, BR), jnp.float32),
            jax.ShapeDtypeStruct((NB, HEADS, BR), jnp.float32),
        ],
    )(x, W1, att_src1, att_dst1)


def _stage_b0_body(ast, adt, s_h, d_h, wout, asv, adv, sbuf, dbuf, w4):
    c = lax.axis_index("c")
    sid = lax.axis_index("s")
    wid = c * NS + sid
    pltpu.sync_copy(ast.at[pl.ds(0, HEADS * N)], asv.at[pl.ds(0, HEADS * N)])
    pltpu.sync_copy(adt.at[pl.ds(0, HEADS * N)], adv.at[pl.ds(0, HEADS * N)])

    def chunk_body(ch, _):
        base = wid * TPS2 + ch * CH0
        pltpu.sync_copy(s_h.at[pl.ds(base, CH0)], sbuf)
        pltpu.sync_copy(d_h.at[pl.ds(base, CH0)], dbuf)

        def q_body(q, _):
            s16 = sbuf[pl.ds(q * L, L)]
            d16 = dbuf[pl.ds(q * L, L)]
            for hh in range(HEADS):
                off = jnp.full((L,), hh * N, jnp.int32)
                t = (plsc.load_gather(asv, [s16 + off])
                     + plsc.load_gather(adv, [d16 + off]))
                t = jnp.maximum(t, 0.2 * t)
                w4[hh, pl.ds(q * L, L)] = jnp.exp(t)
            return 0

        lax.fori_loop(0, CH0 // L, q_body, 0)
        for hh in range(HEADS):
            pltpu.sync_copy(w4.at[hh], wout.at[pl.ds(hh * EPAD + base, CH0)])
        return 0

    lax.fori_loop(0, NCH0, chunk_body, 0)


def _stage_b0(ast, adt, s_h, d_h):
    mesh = plsc.VectorSubcoreMesh(core_axis_name="c", subcore_axis_name="s",
                                  num_cores=NC, num_subcores=NS)
    f = pl.kernel(
        _stage_b0_body,
        out_type=jax.ShapeDtypeStruct((HEADS * EPAD,), jnp.float32),
        mesh=mesh,
        compiler_params=pltpu.CompilerParams(needs_layout_passes=False,
                                             use_tc_tiling_on_sc=False),
        scratch_types=[
            pltpu.VMEM((HEADS * N + L,), jnp.float32),
            pltpu.VMEM((HEADS * N + L,), jnp.float32),
            pltpu.VMEM((CH0,), jnp.int32),
            pltpu.VMEM((CH0,), jnp.int32),
            pltpu.VMEM((HEADS, CH0), jnp.float32),
        ],
    )
    return f(ast, adt, s_h, d_h)


def _zero_acc(acc, zbuf, sid, cols):
    zv = jnp.zeros((L,), jnp.float32)
    for r in range(8):
        for k in range(cols // L):
            zbuf[r, pl.ds(k * L, L)] = zv
    base = sid * SPT
    for r in range(SPT // 8):
        pltpu.sync_copy(zbuf, acc.at[pl.ds(base + r * 8, 8)])


def _stage_b_body(h1h, s_h, d_h, w_h, out1,
                  acc, sbuf, dbuf, wbuf, gidx0, gidx1, rows0, rows1, scat,
                  zbuf, sem0, sem1):
    c = lax.axis_index("c")
    sid = lax.axis_index("s")
    iot = lax.iota(jnp.int32, L)

    for hp in range(2):
        hglob = 2 * c + hp
        hoff = hglob * N
        _zero_acc(acc, zbuf, sid, C1)
        plsc.subcore_barrier()

        def chunk_body(ch, _):
            base = sid * TPS1 + ch * CH
            pltpu.sync_copy(s_h.at[pl.ds(base, CH)], sbuf)
            pltpu.sync_copy(d_h.at[pl.ds(base, CH)], dbuf)
            pltpu.sync_copy(w_h.at[pl.ds(hglob * EPAD + base, CH)], wbuf)

            def fire(g, gidx, rows, sem):
                for q in range(GB // L):
                    s16 = sbuf[pl.ds(g * GB + q * L, L)]
                    gidx[pl.ds(q * L, L)] = s16 + hoff
                pltpu.async_copy(h1h.at[gidx], rows, sem)

            def drain(g, gidx, rows, sem):
                pltpu.make_async_copy(h1h.at[gidx], rows, sem).wait()

                @plsc.parallel_loop(0, GB, unroll=4)
                def scale_body(j):
                    wsp = plsc.load_gather(
                        wbuf, [jnp.full((L,), 0, jnp.int32) + (g * GB + j)])
                    for k in range(HID // L):
                        scat[j, pl.ds(k * L, L)] = rows[j, pl.ds(k * L, L)] * wsp
                    scat[j, pl.ds(HID, L)] = jnp.where(iot == 0, wsp, 0.0)
                pltpu.sync_copy(scat, acc.at[dbuf.at[pl.ds(g * GB, GB)]],
                                add=True)

            fire(0, gidx0, rows0, sem0)

            def pair_body(pr, _):
                g = pr * 2
                fire(g + 1, gidx1, rows1, sem1)
                drain(g, gidx0, rows0, sem0)
                fire(g + 2, gidx0, rows0, sem0)
                drain(g + 1, gidx1, rows1, sem1)
                return 0

            lax.fori_loop(0, NPAIR - 1, pair_body, 0)
            gl = (NPAIR - 1) * 2
            fire(gl + 1, gidx1, rows1, sem1)
            drain(gl, gidx0, rows0, sem0)
            drain(gl + 1, gidx1, rows1, sem1)
            return 0

        lax.fori_loop(0, NCH1, chunk_body, 0)
        plsc.subcore_barrier()
        wb = sid * SPT
        pltpu.sync_copy(acc.at[pl.ds(wb, SPT)],
                        out1.at[hglob].at[pl.ds(wb, SPT)])
        plsc.subcore_barrier()


def _stage_b(h1h, s_h, d_h, w_h):
    mesh = plsc.VectorSubcoreMesh(core_axis_name="c", subcore_axis_name="s",
                                  num_cores=NC, num_subcores=NS)
    f = pl.kernel(
        _stage_b_body,
        out_type=jax.ShapeDtypeStruct((HEADS, ACC_R, C1), jnp.float32),
        mesh=mesh,
        compiler_params=pltpu.CompilerParams(needs_layout_passes=False,
                                             use_tc_tiling_on_sc=False),
        scratch_types=[
            pltpu.VMEM_SHARED((ACC_R, C1), jnp.float32),
            pltpu.VMEM((CH,), jnp.int32),
            pltpu.VMEM((CH,), jnp.int32),
            pltpu.VMEM((CH,), jnp.float32),
            pltpu.VMEM((GB,), jnp.int32),
            pltpu.VMEM((GB,), jnp.int32),
            pltpu.VMEM((GB, HID), jnp.float32),
            pltpu.VMEM((GB, HID), jnp.float32),
            pltpu.VMEM((GB, C1), jnp.float32),
            pltpu.VMEM((8, C1), jnp.float32),
            pltpu.SemaphoreType.DMA,
            pltpu.SemaphoreType.DMA,
        ],
    )
    return f(h1h, s_h, d_h, w_h)


def _stage_c_body(o1_ref, b1_ref, w2_ref, as2w_ref, ad2w_ref,
                  h2_ref, as2_ref, ad2_ref):
    h2 = jnp.zeros((BR, OUT_CH), jnp.float32)
    for hh in range(HEADS):
        m = o1_ref[hh, :, 0:HID]
        dn = o1_ref[hh, :, HID:HID + 1]
        a = m / (dn + _EPS) + b1_ref[0:1, hh * HID:(hh + 1) * HID]
        act = jnp.where(a > 0, a, jnp.exp(a) - 1.0)
        h2 = h2 + lax.dot_general(
            act, w2_ref[pl.ds(hh * HID, HID), :], (((1,), (0,)), ((), ())),
            preferred_element_type=jnp.float32)
    h2_ref[:, 0:OUT_CH] = h2
    h2_ref[:, OUT_CH:IN_CH] = jnp.zeros((BR, IN_CH - OUT_CH), jnp.float32)
    as2_ref[0] = lax.dot_general(as2w_ref[...], h2, (((1,), (1,)), ((), ())),
                                 preferred_element_type=jnp.float32)
    ad2_ref[0] = lax.dot_general(ad2w_ref[...], h2, (((1,), (1,)), ((), ())),
                                 preferred_element_type=jnp.float32)


def _stage_c(out1, b1, W2, att_src2, att_dst2):
    return pl.pallas_call(
        _stage_c_body,
        grid=(NB,),
        in_specs=[
            pl.BlockSpec((HEADS, BR, C1), lambda i: (0, i, 0)),
            pl.BlockSpec((1, HEADS * HID), lambda i: (0, 0)),
            pl.BlockSpec((HEADS * HID, OUT_CH), lambda i: (0, 0)),
            pl.BlockSpec((1, OUT_CH), lambda i: (0, 0)),
            pl.BlockSpec((1, OUT_CH), lambda i: (0, 0)),
        ],
        out_specs=[
            pl.BlockSpec((BR, IN_CH), lambda i: (i, 0)),
            pl.BlockSpec((1, 1, BR), lambda i: (i, 0, 0)),
            pl.BlockSpec((1, 1, BR), lambda i: (i, 0, 0)),
        ],
        out_shape=[
            jax.ShapeDtypeStruct((N, IN_CH), jnp.float32),
            jax.ShapeDtypeStruct((NB, 1, BR), jnp.float32),
            jax.ShapeDtypeStruct((NB, 1, BR), jnp.float32),
        ],
    )(out1, b1, W2, att_src2, att_dst2)


def _stage_d_body(h2p, as2, ad2, s_h, d_h, out2,
                  acc, asv, adv, sbuf, dbuf, wbuf, rows0, rows1, scat, zbuf,
                  sem0, sem1):
    c = lax.axis_index("c")
    sid = lax.axis_index("s")
    wid = c * NS + sid
    iot = lax.iota(jnp.int32, L)
    ebase = wid * TPS2

    pltpu.sync_copy(s_h.at[pl.ds(ebase, TPS2)], sbuf)
    pltpu.sync_copy(d_h.at[pl.ds(ebase, TPS2)], dbuf)
    pltpu.sync_copy(as2.at[pl.ds(0, N)], asv.at[pl.ds(0, N)])
    pltpu.sync_copy(ad2.at[pl.ds(0, N)], adv.at[pl.ds(0, N)])
    _zero_acc(acc, zbuf, sid, C2)
    plsc.subcore_barrier()

    def fire(g, rows, sem):
        pltpu.async_copy(h2p.at[sbuf.at[pl.ds(g * GBD, GBD)]], rows, sem)

    def drain(g, rows, sem):
        pltpu.make_async_copy(h2p.at[sbuf.at[pl.ds(g * GBD, GBD)]], rows,
                              sem).wait()
        for q in range(GBD // L):
            s16 = sbuf[pl.ds(g * GBD + q * L, L)]
            d16 = dbuf[pl.ds(g * GBD + q * L, L)]
            t = plsc.load_gather(asv, [s16]) + plsc.load_gather(adv, [d16])
            t = jnp.maximum(t, 0.2 * t)
            wbuf[pl.ds(q * L, L)] = jnp.exp(t)

        @plsc.parallel_loop(0, GBD, unroll=4)
        def scale_body(j):
            wsp = plsc.load_gather(wbuf, [jnp.full((L,), 0, jnp.int32) + j])
            for k in range(OUT_CH // L):
                scat[j, pl.ds(k * L, L)] = rows[j, pl.ds(k * L, L)] * wsp
            scat[j, pl.ds(OUT_CH, L)] = jnp.where(iot == 0, wsp, 0.0)
        pltpu.sync_copy(scat, acc.at[dbuf.at[pl.ds(g * GBD, GBD)]], add=True)

    fire(0, rows0, sem0)

    def pair_body(pr, _):
        g = pr * 2
        fire(g + 1, rows1, sem1)
        drain(g, rows0, sem0)
        fire(g + 2, rows0, sem0)
        drain(g + 1, rows1, sem1)
        return 0

    lax.fori_loop(0, NPAIRD - 1, pair_body, 0)
    gl = (NPAIRD - 1) * 2
    fire(gl + 1, rows1, sem1)
    drain(gl, rows0, sem0)
    drain(gl + 1, rows1, sem1)
    plsc.subcore_barrier()
    wb = sid * SPT
    pltpu.sync_copy(acc.at[pl.ds(wb, SPT)], out2.at[c].at[pl.ds(wb, SPT)])


def _stage_d(h2p, as2, ad2, s_h, d_h):
    mesh = plsc.VectorSubcoreMesh(core_axis_name="c", subcore_axis_name="s",
                                  num_cores=NC, num_subcores=NS)
    f = pl.kernel(
        _stage_d_body,
        out_type=jax.ShapeDtypeStruct((NC, ACC_R, C2), jnp.float32),
        mesh=mesh,
        compiler_params=pltpu.CompilerParams(needs_layout_passes=False,
                                             use_tc_tiling_on_sc=False),
        scratch_types=[
            pltpu.VMEM_SHARED((ACC_R, C2), jnp.float32),
            pltpu.VMEM((N + L,), jnp.float32),
            pltpu.VMEM((N + L,), jnp.float32),
            pltpu.VMEM((TPS2,), jnp.int32),
            pltpu.VMEM((TPS2,), jnp.int32),
            pltpu.VMEM((GBD,), jnp.float32),
            pltpu.VMEM((GBD, IN_CH), jnp.float32),
            pltpu.VMEM((GBD, IN_CH), jnp.float32),
            pltpu.VMEM((GBD, C2), jnp.float32),
            pltpu.VMEM((8, C2), jnp.float32),
            pltpu.SemaphoreType.DMA,
            pltpu.SemaphoreType.DMA,
        ],
    )
    return f(h2p, as2, ad2, s_h, d_h)


def _stage_e_body(o2_ref, b2_ref, out_ref):
    m = o2_ref[0, :, 0:OUT_CH] + o2_ref[1, :, 0:OUT_CH]
    dn = o2_ref[0, :, OUT_CH:OUT_CH + 1] + o2_ref[1, :, OUT_CH:OUT_CH + 1]
    o = m / (dn + _EPS) + b2_ref[...]
    mx = jnp.max(o, axis=1, keepdims=True)
    e = jnp.exp(o - mx)
    s = jnp.sum(e, axis=1, keepdims=True)
    out_ref[...] = (o - mx) - jnp.log(s)


def _stage_e(out2, b2):
    return pl.pallas_call(
        _stage_e_body,
        grid=(NB,),
        in_specs=[
            pl.BlockSpec((NC, BR, C2), lambda i: (0, i, 0)),
            pl.BlockSpec((1, OUT_CH), lambda i: (0, 0)),
        ],
        out_specs=pl.BlockSpec((BR, OUT_CH), lambda i: (i, 0)),
        out_shape=jax.ShapeDtypeStruct((N, OUT_CH), jnp.float32),
    )(out2, b2)


def kernel(x, edge_index, W1, att_src1, att_dst1, b1, W2, att_src2, att_dst2, b2):
    src = edge_index[0]
    dst = edge_index[1]
    loop = jnp.arange(N, dtype=jnp.int32)
    pad = EPAD - E1
    s = jnp.concatenate([src, loop, jnp.zeros((pad,), jnp.int32)])
    d = jnp.concatenate([dst, loop, jnp.full((pad,), N, jnp.int32)])

    h1h, ast, adt = _stage_a(x, W1, att_src1, att_dst1)
    h1h = (h1h.reshape(NB, HEADS, BR, HID)
           .transpose(1, 0, 2, 3).reshape(HEADS * N, HID))
    ast = ast.transpose(1, 0, 2).reshape(HEADS * N)
    adt = adt.transpose(1, 0, 2).reshape(HEADS * N)
    w_h = _stage_b0(ast, adt, s, d)
    out1 = _stage_b(h1h, s, d, w_h)
    h2p, as2, ad2 = _stage_c(out1, b1.reshape(1, -1), W2, att_src2, att_dst2)
    out2 = _stage_d(h2p, as2.reshape(N), ad2.reshape(N), s, d)
    return _stage_e(out2, b2.reshape(1, -1))


# speedup vs baseline: 30.5477x; 1.0015x over previous
import jax
import jax.numpy as jnp
from jax import lax
from jax.experimental import pallas as pl
from jax.experimental.pallas import tpu as pltpu
from jax.experimental.pallas import tpu_sc as plsc

N = 10000
E_RAW = 320000
E1 = E_RAW + N
IN_CH = 128
HID = 128
HEADS = 4
OUT_CH = 64

NC = 2
NS = 16
L = 16

EPAD = 331776
TPS1 = EPAD // NS
TPS2 = EPAD // (NC * NS)

CH = 1152
NCH1 = TPS1 // CH
CH0 = 576
NCH0 = TPS2 // CH0
GB = 64
NGC = CH // GB
NPAIR = NGC // 2
GBD = 96
NGD = TPS2 // GBD
NPAIRD = NGD // 2

ACC_R = 10112
SPT = ACC_R // NS
C1 = 144
C2 = 80

NB = 25
BR = N // NB

_EPS = 1e-30


def _stage_a_body(x_ref, w1_ref, asrc_ref, adst_ref, h1h_ref, as_ref, ad_ref):
    xb = x_ref[...]
    h = lax.dot_general(xb, w1_ref[...], (((1,), (0,)), ((), ())),
                        preferred_element_type=jnp.float32)
    for hh in range(HEADS):
        seg = h[:, hh * HID:(hh + 1) * HID]
        h1h_ref[pl.ds(hh * BR, BR), :] = seg
        as_ref[0, pl.ds(hh, 1), :] = lax.dot_general(
            asrc_ref[pl.ds(hh, 1), :], seg, (((1,), (1,)), ((), ())),
            preferred_element_type=jnp.float32)
        ad_ref[0, pl.ds(hh, 1), :] = lax.dot_general(
            adst_ref[pl.ds(hh, 1), :], seg, (((1,), (1,)), ((), ())),
            preferred_element_type=jnp.float32)


def _stage_a(x, W1, att_src1, att_dst1):
    return pl.pallas_call(
        _stage_a_body,
        grid=(NB,),
        in_specs=[
            pl.BlockSpec((BR, IN_CH), lambda i: (i, 0)),
            pl.BlockSpec((IN_CH, HEADS * HID), lambda i: (0, 0)),
            pl.BlockSpec((HEADS, HID), lambda i: (0, 0)),
            pl.BlockSpec((HEADS, HID), lambda i: (0, 0)),
        ],
        out_specs=[
            pl.BlockSpec((HEADS * BR, HID), lambda i: (i, 0)),
            pl.BlockSpec((1, HEADS, BR), lambda i: (i, 0, 0)),
            pl.BlockSpec((1, HEADS, BR), lambda i: (i, 0, 0)),
        ],
        out_shape=[
            jax.ShapeDtypeStruct((NB * HEADS * BR, HID), jnp.float32),
            jax.ShapeDtypeStruct((NB, HEADS, BR), jnp.float32),
            jax.ShapeDtypeStruct((NB, HEADS, BR), jnp.float32),
        ],
    )(x, W1, att_src1, att_dst1)


def _stage_b0_body(ast, adt, s_h, d_h, wout, asv, adv, sbuf, dbuf, w4):
    c = lax.axis_index("c")
    sid = lax.axis_index("s")
    wid = c * NS + sid
    pltpu.sync_copy(ast.at[pl.ds(0, HEADS * N)], asv.at[pl.ds(0, HEADS * N)])
    pltpu.sync_copy(adt.at[pl.ds(0, HEADS * N)], adv.at[pl.ds(0, HEADS * N)])

    def chunk_body(ch, _):
        base = wid * TPS2 + ch * CH0
        pltpu.sync_copy(s_h.at[pl.ds(base, CH0)], sbuf)
        pltpu.sync_copy(d_h.at[pl.ds(base, CH0)], dbuf)

        def q_body(q, _):
            s16 = sbuf[pl.ds(q * L, L)]
            d16 = dbuf[pl.ds(q * L, L)]
            for hh in range(HEADS):
                off = jnp.full((L,), hh * N, jnp.int32)
                t = (plsc.load_gather(asv, [s16 + off])
                     + plsc.load_gather(adv, [d16 + off]))
                t = jnp.maximum(t, 0.2 * t)
                w4[hh, pl.ds(q * L, L)] = jnp.exp(t)
            return 0

        lax.fori_loop(0, CH0 // L, q_body, 0)
        for hh in range(HEADS):
            pltpu.sync_copy(w4.at[hh], wout.at[pl.ds(hh * EPAD + base, CH0)])
        return 0

    lax.fori_loop(0, NCH0, chunk_body, 0)


def _stage_b0(ast, adt, s_h, d_h):
    mesh = plsc.VectorSubcoreMesh(core_axis_name="c", subcore_axis_name="s",
                                  num_cores=NC, num_subcores=NS)
    f = pl.kernel(
        _stage_b0_body,
        out_type=jax.ShapeDtypeStruct((HEADS * EPAD,), jnp.float32),
        mesh=mesh,
        compiler_params=pltpu.CompilerParams(needs_layout_passes=False,
                                             use_tc_tiling_on_sc=False),
        scratch_types=[
            pltpu.VMEM((HEADS * N + L,), jnp.float32),
            pltpu.VMEM((HEADS * N + L,), jnp.float32),
            pltpu.VMEM((CH0,), jnp.int32),
            pltpu.VMEM((CH0,), jnp.int32),
            pltpu.VMEM((HEADS, CH0), jnp.float32),
        ],
    )
    return f(ast, adt, s_h, d_h)


def _zero_acc(acc, zbuf, sid, cols):
    zv = jnp.zeros((L,), jnp.float32)
    for r in range(8):
        for k in range(cols // L):
            zbuf[r, pl.ds(k * L, L)] = zv
    base = sid * SPT
    for r in range(SPT // 8):
        pltpu.sync_copy(zbuf, acc.at[pl.ds(base + r * 8, 8)])


def _stage_b_body(h1h, s_h, d_h, w_h, out1,
                  acc, sbuf, dbuf, wbuf, gidx0, gidx1, rows0, rows1, scat,
                  zbuf, sem0, sem1):
    c = lax.axis_index("c")
    sid = lax.axis_index("s")
    iot = lax.iota(jnp.int32, L)

    for hp in range(2):
        hglob = 2 * c + hp
        hoff = hglob * N
        _zero_acc(acc, zbuf, sid, C1)
        plsc.subcore_barrier()

        def chunk_body(ch, _):
            base = sid * TPS1 + ch * CH
            pltpu.sync_copy(s_h.at[pl.ds(base, CH)], sbuf)
            pltpu.sync_copy(d_h.at[pl.ds(base, CH)], dbuf)
            pltpu.sync_copy(w_h.at[pl.ds(hglob * EPAD + base, CH)], wbuf)

            def fire(g, gidx, rows, sem):
                for q in range(GB // L):
                    s16 = sbuf[pl.ds(g * GB + q * L, L)]
                    gidx[pl.ds(q * L, L)] = s16 + hoff
                pltpu.async_copy(h1h.at[gidx], rows, sem)

            def drain(g, gidx, rows, sem):
                pltpu.make_async_copy(h1h.at[gidx], rows, sem).wait()

                @plsc.parallel_loop(0, GB, unroll=8)
                def scale_body(j):
                    wsp = plsc.load_gather(
                        wbuf, [jnp.full((L,), 0, jnp.int32) + (g * GB + j)])
                    for k in range(HID // L):
                        scat[j, pl.ds(k * L, L)] = rows[j, pl.ds(k * L, L)] * wsp
                    scat[j, pl.ds(HID, L)] = jnp.where(iot == 0, wsp, 0.0)
                pltpu.sync_copy(scat, acc.at[dbuf.at[pl.ds(g * GB, GB)]],
                                add=True)

            fire(0, gidx0, rows0, sem0)

            def pair_body(pr, _):
                g = pr * 2
                fire(g + 1, gidx1, rows1, sem1)
                drain(g, gidx0, rows0, sem0)
                fire(g + 2, gidx0, rows0, sem0)
                drain(g + 1, gidx1, rows1, sem1)
                return 0

            lax.fori_loop(0, NPAIR - 1, pair_body, 0)
            gl = (NPAIR - 1) * 2
            fire(gl + 1, gidx1, rows1, sem1)
            drain(gl, gidx0, rows0, sem0)
            drain(gl + 1, gidx1, rows1, sem1)
            return 0

        lax.fori_loop(0, NCH1, chunk_body, 0)
        plsc.subcore_barrier()
        wb = sid * SPT
        pltpu.sync_copy(acc.at[pl.ds(wb, SPT)],
                        out1.at[hglob].at[pl.ds(wb, SPT)])
        plsc.subcore_barrier()


def _stage_b(h1h, s_h, d_h, w_h):
    mesh = plsc.VectorSubcoreMesh(core_axis_name="c", subcore_axis_name="s",
                                  num_cores=NC, num_subcores=NS)
    f = pl.kernel(
        _stage_b_body,
        out_type=jax.ShapeDtypeStruct((HEADS, ACC_R, C1), jnp.float32),
        mesh=mesh,
        compiler_params=pltpu.CompilerParams(needs_layout_passes=False,
                                             use_tc_tiling_on_sc=False),
        scratch_types=[
            pltpu.VMEM_SHARED((ACC_R, C1), jnp.float32),
            pltpu.VMEM((CH,), jnp.int32),
            pltpu.VMEM((CH,), jnp.int32),
            pltpu.VMEM((CH,), jnp.float32),
            pltpu.VMEM((GB,), jnp.int32),
            pltpu.VMEM((GB,), jnp.int32),
            pltpu.VMEM((GB, HID), jnp.float32),
            pltpu.VMEM((GB, HID), jnp.float32),
            pltpu.VMEM((GB, C1), jnp.float32),
            pltpu.VMEM((8, C1), jnp.float32),
            pltpu.SemaphoreType.DMA,
            pltpu.SemaphoreType.DMA,
        ],
    )
    return f(h1h, s_h, d_h, w_h)


def _stage_c_body(o1_ref, b1_ref, w2_ref, as2w_ref, ad2w_ref,
                  h2_ref, as2_ref, ad2_ref):
    h2 = jnp.zeros((BR, OUT_CH), jnp.float32)
    for hh in range(HEADS):
        m = o1_ref[hh, :, 0:HID]
        dn = o1_ref[hh, :, HID:HID + 1]
        a = m / (dn + _EPS) + b1_ref[0:1, hh * HID:(hh + 1) * HID]
        act = jnp.where(a > 0, a, jnp.exp(a) - 1.0)
        h2 = h2 + lax.dot_general(
            act, w2_ref[pl.ds(hh * HID, HID), :], (((1,), (0,)), ((), ())),
            preferred_element_type=jnp.float32)
    h2_ref[:, 0:OUT_CH] = h2
    h2_ref[:, OUT_CH:IN_CH] = jnp.zeros((BR, IN_CH - OUT_CH), jnp.float32)
    as2_ref[0] = lax.dot_general(as2w_ref[...], h2, (((1,), (1,)), ((), ())),
                                 preferred_element_type=jnp.float32)
    ad2_ref[0] = lax.dot_general(ad2w_ref[...], h2, (((1,), (1,)), ((), ())),
                                 preferred_element_type=jnp.float32)


def _stage_c(out1, b1, W2, att_src2, att_dst2):
    return pl.pallas_call(
        _stage_c_body,
        grid=(NB,),
        in_specs=[
            pl.BlockSpec((HEADS, BR, C1), lambda i: (0, i, 0)),
            pl.BlockSpec((1, HEADS * HID), lambda i: (0, 0)),
            pl.BlockSpec((HEADS * HID, OUT_CH), lambda i: (0, 0)),
            pl.BlockSpec((1, OUT_CH), lambda i: (0, 0)),
            pl.BlockSpec((1, OUT_CH), lambda i: (0, 0)),
        ],
        out_specs=[
            pl.BlockSpec((BR, IN_CH), lambda i: (i, 0)),
            pl.BlockSpec((1, 1, BR), lambda i: (i, 0, 0)),
            pl.BlockSpec((1, 1, BR), lambda i: (i, 0, 0)),
        ],
        out_shape=[
            jax.ShapeDtypeStruct((N, IN_CH), jnp.float32),
            jax.ShapeDtypeStruct((NB, 1, BR), jnp.float32),
            jax.ShapeDtypeStruct((NB, 1, BR), jnp.float32),
        ],
    )(out1, b1, W2, att_src2, att_dst2)


def _stage_d_body(h2p, as2, ad2, s_h, d_h, out2,
                  acc, asv, adv, sbuf, dbuf, wbuf, rows0, rows1, scat, zbuf,
                  sem0, sem1):
    c = lax.axis_index("c")
    sid = lax.axis_index("s")
    wid = c * NS + sid
    iot = lax.iota(jnp.int32, L)
    ebase = wid * TPS2

    pltpu.sync_copy(s_h.at[pl.ds(ebase, TPS2)], sbuf)
    pltpu.sync_copy(d_h.at[pl.ds(ebase, TPS2)], dbuf)
    pltpu.sync_copy(as2.at[pl.ds(0, N)], asv.at[pl.ds(0, N)])
    pltpu.sync_copy(ad2.at[pl.ds(0, N)], adv.at[pl.ds(0, N)])
    _zero_acc(acc, zbuf, sid, C2)
    plsc.subcore_barrier()

    def fire(g, rows, sem):
        pltpu.async_copy(h2p.at[sbuf.at[pl.ds(g * GBD, GBD)]], rows, sem)

    def drain(g, rows, sem):
        pltpu.make_async_copy(h2p.at[sbuf.at[pl.ds(g * GBD, GBD)]], rows,
                              sem).wait()
        for q in range(GBD // L):
            s16 = sbuf[pl.ds(g * GBD + q * L, L)]
            d16 = dbuf[pl.ds(g * GBD + q * L, L)]
            t = plsc.load_gather(asv, [s16]) + plsc.load_gather(adv, [d16])
            t = jnp.maximum(t, 0.2 * t)
            wbuf[pl.ds(q * L, L)] = jnp.exp(t)

        @plsc.parallel_loop(0, GBD, unroll=8)
        def scale_body(j):
            wsp = plsc.load_gather(wbuf, [jnp.full((L,), 0, jnp.int32) + j])
            for k in range(OUT_CH // L):
                scat[j, pl.ds(k * L, L)] = rows[j, pl.ds(k * L, L)] * wsp
            scat[j, pl.ds(OUT_CH, L)] = jnp.where(iot == 0, wsp, 0.0)
        pltpu.sync_copy(scat, acc.at[dbuf.at[pl.ds(g * GBD, GBD)]], add=True)

    fire(0, rows0, sem0)

    def pair_body(pr, _):
        g = pr * 2
        fire(g + 1, rows1, sem1)
        drain(g, rows0, sem0)
        fire(g + 2, rows0, sem0)
        drain(g + 1, rows1, sem1)
        return 0

    lax.fori_loop(0, NPAIRD - 1, pair_body, 0)
    gl = (NPAIRD - 1) * 2
    fire(gl + 1, rows1, sem1)
    drain(gl, rows0, sem0)
    drain(gl + 1, rows1, sem1)
    plsc.subcore_barrier()
    wb = sid * SPT
    pltpu.sync_copy(acc.at[pl.ds(wb, SPT)], out2.at[c].at[pl.ds(wb, SPT)])


def _stage_d(h2p, as2, ad2, s_h, d_h):
    mesh = plsc.VectorSubcoreMesh(core_axis_name="c", subcore_axis_name="s",
                                  num_cores=NC, num_subcores=NS)
    f = pl.kernel(
        _stage_d_body,
        out_type=jax.ShapeDtypeStruct((NC, ACC_R, C2), jnp.float32),
        mesh=mesh,
        compiler_params=pltpu.CompilerParams(needs_layout_passes=False,
                                             use_tc_tiling_on_sc=False),
        scratch_types=[
            pltpu.VMEM_SHARED((ACC_R, C2), jnp.float32),
            pltpu.VMEM((N + L,), jnp.float32),
            pltpu.VMEM((N + L,), jnp.float32),
            pltpu.VMEM((TPS2,), jnp.int32),
            pltpu.VMEM((TPS2,), jnp.int32),
            pltpu.VMEM((GBD,), jnp.float32),
            pltpu.VMEM((GBD, IN_CH), jnp.float32),
            pltpu.VMEM((GBD, IN_CH), jnp.float32),
            pltpu.VMEM((GBD, C2), jnp.float32),
            pltpu.VMEM((8, C2), jnp.float32),
            pltpu.SemaphoreType.DMA,
            pltpu.SemaphoreType.DMA,
        ],
    )
    return f(h2p, as2, ad2, s_h, d_h)


def _stage_e_body(o2_ref, b2_ref, out_ref):
    m = o2_ref[0, :, 0:OUT_CH] + o2_ref[1, :, 0:OUT_CH]
    dn = o2_ref[0, :, OUT_CH:OUT_CH + 1] + o2_ref[1, :, OUT_CH:OUT_CH + 1]
    o = m / (dn + _EPS) + b2_ref[...]
    mx = jnp.max(o, axis=1, keepdims=True)
    e = jnp.exp(o - mx)
    s = jnp.sum(e, axis=1, keepdims=True)
    out_ref[...] = (o - mx) - jnp.log(s)


def _stage_e(out2, b2):
    return pl.pallas_call(
        _stage_e_body,
        grid=(NB,),
        in_specs=[
            pl.BlockSpec((NC, BR, C2), lambda i: (0, i, 0)),
            pl.BlockSpec((1, OUT_CH), lambda i: (0, 0)),
        ],
        out_specs=pl.BlockSpec((BR, OUT_CH), lambda i: (i, 0)),
        out_shape=jax.ShapeDtypeStruct((N, OUT_CH), jnp.float32),
    )(out2, b2)


def kernel(x, edge_index, W1, att_src1, att_dst1, b1, W2, att_src2, att_dst2, b2):
    src = edge_index[0]
    dst = edge_index[1]
    loop = jnp.arange(N, dtype=jnp.int32)
    pad = EPAD - E1
    s = jnp.concatenate([src, loop, jnp.zeros((pad,), jnp.int32)])
    d = jnp.concatenate([dst, loop, jnp.full((pad,), N, jnp.int32)])

    h1h, ast, adt = _stage_a(x, W1, att_src1, att_dst1)
    h1h = (h1h.reshape(NB, HEADS, BR, HID)
           .transpose(1, 0, 2, 3).reshape(HEADS * N, HID))
    ast = ast.transpose(1, 0, 2).reshape(HEADS * N)
    adt = adt.transpose(1, 0, 2).reshape(HEADS * N)
    w_h = _stage_b0(ast, adt, s, d)
    out1 = _stage_b(h1h, s, d, w_h)
    h2p, as2, ad2 = _stage_c(out1, b1.reshape(1, -1), W2, att_src2, att_dst2)
    out2 = _stage_d(h2p, as2.reshape(N), ad2.reshape(N), s, d)
    return _stage_e(out2, b2.reshape(1, -1))
